# initial kernel scaffold (unmeasured)
import functools

import jax
import jax.numpy as jnp
from jax import lax
from jax.experimental import pallas as pl
from jax.experimental.pallas import tpu as pltpu

N_DEV = 4
SQ = 2048
SKV_SHARD = 2048
HQ = 8
DH = 128
DM = 1024
QB = 256
SCALE = 0.08838834764831843


def _q_body(x_ref, wq_ref, q_ref):
    q_ref[...] = jnp.dot(
        x_ref[...], wq_ref[...], preferred_element_type=jnp.float32
    )


def _attn_body(q_ref, k_ref, v_ref, o_ref, l_ref):
    my = lax.axis_index("i")
    qb = pl.program_id(1)
    q = q_ref[:, 0, :]
    k = k_ref[:, 0, :]
    v = v_ref[:, 0, :]
    s = lax.dot_general(
        q, k, (((1,), (1,)), ((), ())), preferred_element_type=jnp.float32
    ) * SCALE
    qi = qb * QB + lax.broadcasted_iota(jnp.int32, (QB, SKV_SHARD), 0)
    ki = my * SKV_SHARD + lax.broadcasted_iota(jnp.int32, (QB, SKV_SHARD), 1)
    mask = (jnp.abs(qi - ki) <= 128) | (ki < 32) | (qi < 32)
    w = jnp.where(mask, jnp.exp(s), 0.0)
    o_ref[:, 0, :] = jnp.dot(w, v, preferred_element_type=jnp.float32)
    l_ref[...] = jnp.sum(w, axis=1, keepdims=True)


def _comm_body(
    o_ref, l_ref, wo_ref, out_ref,
    comm_o, comm_l, acc_o, acc_l,
    send_o, recv_o, send_l, recv_l,
):
    my = lax.axis_index("i")
    left = (my + N_DEV - 1) % N_DEV
    right = (my + 1) % N_DEV

    barrier_sem = pltpu.get_barrier_semaphore()
    for nbr in [left, right]:
        pl.semaphore_signal(
            barrier_sem, inc=1,
            device_id=(nbr,), device_id_type=pl.DeviceIdType.MESH,
        )
    pl.semaphore_wait(barrier_sem, 2)

    acc_o[...] = o_ref[...]
    acc_l[...] = l_ref[...]
    comm_o[0] = o_ref[...]
    comm_l[0] = l_ref[...]

    for h in range(N_DEV - 1):
        ss, rs = h % 2, (h + 1) % 2
        rdma_o = pltpu.make_async_remote_copy(
            src_ref=comm_o.at[ss], dst_ref=comm_o.at[rs],
            send_sem=send_o.at[ss], recv_sem=recv_o.at[rs],
            device_id=(right,), device_id_type=pl.DeviceIdType.MESH,
        )
        rdma_l = pltpu.make_async_remote_copy(
            src_ref=comm_l.at[ss], dst_ref=comm_l.at[rs],
            send_sem=send_l.at[ss], recv_sem=recv_l.at[rs],
            device_id=(right,), device_id_type=pl.DeviceIdType.MESH,
        )
        rdma_o.start()
        rdma_l.start()
        rdma_o.wait()
        rdma_l.wait()
        acc_o[...] += comm_o[rs]
        acc_l[...] += comm_l[rs]

    ctx = acc_o[...] / acc_l[...][:, :, None]
    out_ref[...] = jnp.dot(
        ctx.reshape(SQ, DM), wo_ref[...], preferred_element_type=jnp.float32
    )


def kernel(x, Wq, K_ext, V_ext, Wo):
    x2 = x[0]
    K = K_ext[0]
    V = V_ext[0]

    q = pl.pallas_call(
        _q_body,
        out_shape=jax.ShapeDtypeStruct((SQ, DM), jnp.float32),
        in_specs=[
            pl.BlockSpec(memory_space=pltpu.VMEM),
            pl.BlockSpec(memory_space=pltpu.VMEM),
        ],
        out_specs=pl.BlockSpec(memory_space=pltpu.VMEM),
    )(x2, Wq)
    q3 = q.reshape(SQ, HQ, DH)

    o, l = pl.pallas_call(
        _attn_body,
        grid=(HQ, SQ // QB),
        in_specs=[
            pl.BlockSpec((QB, 1, DH), lambda h, qb: (qb, h, 0)),
            pl.BlockSpec((SKV_SHARD, 1, DH), lambda h, qb: (0, h, 0)),
            pl.BlockSpec((SKV_SHARD, 1, DH), lambda h, qb: (0, h, 0)),
        ],
        out_shape=[
            jax.ShapeDtypeStruct((SQ, HQ, DH), jnp.float32),
            jax.ShapeDtypeStruct((SQ, HQ), jnp.float32),
        ],
        out_specs=[
            pl.BlockSpec((QB, 1, DH), lambda h, qb: (qb, h, 0)),
            pl.BlockSpec((QB, 1), lambda h, qb: (qb, h)),
        ],
    )(q3, K, V)

    out = pl.pallas_call(
        _comm_body,
        out_shape=jax.ShapeDtypeStruct((SQ, DM), jnp.float32),
        in_specs=[
            pl.BlockSpec(memory_space=pltpu.VMEM),
            pl.BlockSpec(memory_space=pltpu.VMEM),
            pl.BlockSpec(memory_space=pltpu.VMEM),
        ],
        out_specs=pl.BlockSpec(memory_space=pltpu.VMEM),
        scratch_shapes=[
            pltpu.VMEM((2, SQ, HQ, DH), jnp.float32),
            pltpu.VMEM((2, SQ, HQ), jnp.float32),
            pltpu.VMEM((SQ, HQ, DH), jnp.float32),
            pltpu.VMEM((SQ, HQ), jnp.float32),
            pltpu.SemaphoreType.DMA((2,)),
            pltpu.SemaphoreType.DMA((2,)),
            pltpu.SemaphoreType.DMA((2,)),
            pltpu.SemaphoreType.DMA((2,)),
        ],
        compiler_params=pltpu.CompilerParams(collective_id=0),
    )(o, l, Wo)

    return out.reshape(1, SQ, DM)


# baseline (device time: 501375 ns/iter reference)
import functools

import jax
import jax.numpy as jnp
from jax import lax
from jax.experimental import pallas as pl
from jax.experimental.pallas import tpu as pltpu

N_DEV = 4
SQ = 2048
SKV_SHARD = 2048
HQ = 8
DH = 128
DM = 1024
QB = 256
SCALE = 0.08838834764831843


def _q_body(x_ref, wq_ref, q_ref):
    q_ref[...] = jnp.dot(
        x_ref[...], wq_ref[...], preferred_element_type=jnp.float32
    )


def _attn_body(q_ref, k_ref, v_ref, o_ref, l_ref):
    my = lax.axis_index("i")
    qb = pl.program_id(0)
    qi = qb * QB + lax.broadcasted_iota(jnp.int32, (QB, SKV_SHARD), 0)
    ki = my * SKV_SHARD + lax.broadcasted_iota(jnp.int32, (QB, SKV_SHARD), 1)
    mask = (jnp.abs(qi - ki) <= 128) | (ki < 32) | (qi < 32)
    l_cols = []
    for h in range(HQ):
        q = q_ref[:, h, :]
        k = k_ref[:, h, :]
        v = v_ref[:, h, :]
        s = lax.dot_general(
            q, k, (((1,), (1,)), ((), ())), preferred_element_type=jnp.float32
        ) * SCALE
        w = jnp.where(mask, jnp.exp(s), 0.0)
        o_ref[:, h, :] = jnp.dot(w, v, preferred_element_type=jnp.float32)
        l_cols.append(jnp.sum(w, axis=1, keepdims=True))
    l_ref[...] = jnp.concatenate(l_cols, axis=1)


def _comm_body(
    o_ref, l_ref, wo_ref, out_ref,
    comm_o, comm_l, acc_o, acc_l,
    send_o, recv_o, send_l, recv_l,
):
    my = lax.axis_index("i")
    left = (my + N_DEV - 1) % N_DEV
    right = (my + 1) % N_DEV

    barrier_sem = pltpu.get_barrier_semaphore()
    for nbr in [left, right]:
        pl.semaphore_signal(
            barrier_sem, inc=1,
            device_id=(nbr,), device_id_type=pl.DeviceIdType.MESH,
        )
    pl.semaphore_wait(barrier_sem, 2)

    acc_o[...] = o_ref[...]
    acc_l[...] = l_ref[...]
    comm_o[0] = o_ref[...]
    comm_l[0] = l_ref[...]

    for h in range(N_DEV - 1):
        ss, rs = h % 2, (h + 1) % 2
        rdma_o = pltpu.make_async_remote_copy(
            src_ref=comm_o.at[ss], dst_ref=comm_o.at[rs],
            send_sem=send_o.at[ss], recv_sem=recv_o.at[rs],
            device_id=(right,), device_id_type=pl.DeviceIdType.MESH,
        )
        rdma_l = pltpu.make_async_remote_copy(
            src_ref=comm_l.at[ss], dst_ref=comm_l.at[rs],
            send_sem=send_l.at[ss], recv_sem=recv_l.at[rs],
            device_id=(right,), device_id_type=pl.DeviceIdType.MESH,
        )
        rdma_o.start()
        rdma_l.start()
        rdma_o.wait()
        rdma_l.wait()
        acc_o[...] += comm_o[rs]
        acc_l[...] += comm_l[rs]

    ctx = acc_o[...] / acc_l[...][:, :, None]
    out_ref[...] = jnp.dot(
        ctx.reshape(SQ, DM), wo_ref[...], preferred_element_type=jnp.float32
    )


def kernel(x, Wq, K_ext, V_ext, Wo):
    x2 = x[0]
    K = K_ext[0]
    V = V_ext[0]

    q = pl.pallas_call(
        _q_body,
        out_shape=jax.ShapeDtypeStruct((SQ, DM), jnp.float32),
        in_specs=[
            pl.BlockSpec(memory_space=pltpu.VMEM),
            pl.BlockSpec(memory_space=pltpu.VMEM),
        ],
        out_specs=pl.BlockSpec(memory_space=pltpu.VMEM),
    )(x2, Wq)
    q3 = q.reshape(SQ, HQ, DH)

    o, l = pl.pallas_call(
        _attn_body,
        grid=(SQ // QB,),
        in_specs=[
            pl.BlockSpec((QB, HQ, DH), lambda qb: (qb, 0, 0)),
            pl.BlockSpec((SKV_SHARD, HQ, DH), lambda qb: (0, 0, 0)),
            pl.BlockSpec((SKV_SHARD, HQ, DH), lambda qb: (0, 0, 0)),
        ],
        out_shape=[
            jax.ShapeDtypeStruct((SQ, HQ, DH), jnp.float32),
            jax.ShapeDtypeStruct((SQ, HQ), jnp.float32),
        ],
        out_specs=[
            pl.BlockSpec((QB, HQ, DH), lambda qb: (qb, 0, 0)),
            pl.BlockSpec((QB, HQ), lambda qb: (qb, 0)),
        ],
        compiler_params=pltpu.CompilerParams(
            vmem_limit_bytes=100 * 1024 * 1024,
        ),
    )(q3, K, V)

    out = pl.pallas_call(
        _comm_body,
        out_shape=jax.ShapeDtypeStruct((SQ, DM), jnp.float32),
        in_specs=[
            pl.BlockSpec(memory_space=pltpu.VMEM),
            pl.BlockSpec(memory_space=pltpu.VMEM),
            pl.BlockSpec(memory_space=pltpu.VMEM),
        ],
        out_specs=pl.BlockSpec(memory_space=pltpu.VMEM),
        scratch_shapes=[
            pltpu.VMEM((2, SQ, HQ, DH), jnp.float32),
            pltpu.VMEM((2, SQ, HQ), jnp.float32),
            pltpu.VMEM((SQ, HQ, DH), jnp.float32),
            pltpu.VMEM((SQ, HQ), jnp.float32),
            pltpu.SemaphoreType.DMA((2,)),
            pltpu.SemaphoreType.DMA((2,)),
            pltpu.SemaphoreType.DMA((2,)),
            pltpu.SemaphoreType.DMA((2,)),
        ],
        compiler_params=pltpu.CompilerParams(
            collective_id=0, vmem_limit_bytes=100 * 1024 * 1024,
        ),
    )(o, l, Wo)

    return out.reshape(1, SQ, DM)


# device time: 234043 ns/iter; 2.1422x vs baseline; 2.1422x over previous
import functools

import jax
import jax.numpy as jnp
from jax import lax
from jax.experimental import pallas as pl
from jax.experimental.pallas import tpu as pltpu

N_DEV = 4
SQ = 2048
SKV_SHARD = 2048
HQ = 8
DH = 128
DM = 1024
QB = 256
WKV = 512
GKV = 128
NG = 32
SCALE = 0.08838834764831843

CHUNKS = [(32 + 176 * c, 176) for c in range(10)] + [(1792, 128), (1920, 128)]
SLIV0 = 1920


def _q_body(x_ref, wq_ref, q_ref):
    q_ref[...] = jnp.dot(
        x_ref[...], wq_ref[...], preferred_element_type=jnp.float32
    )


def _attn_body(q_ref, k_ref, v_ref, o_ref, l_ref):
    my = lax.axis_index("i")
    qb = pl.program_id(0)
    ws = jnp.clip(qb * QB - 128 - my * SKV_SHARD, 0, SKV_SHARD - WKV)
    qi_w = qb * QB + lax.broadcasted_iota(jnp.int32, (QB, WKV), 0)
    ki_w = my * SKV_SHARD + ws + lax.broadcasted_iota(jnp.int32, (QB, WKV), 1)
    mask_w = (jnp.abs(qi_w - ki_w) <= 128) & (ki_w >= NG)
    ki_g = my * SKV_SHARD + lax.broadcasted_iota(jnp.int32, (QB, GKV), 1)
    mask_g = ki_g < NG
    l_cols = []
    for h in range(HQ):
        q = q_ref[:, h, :]
        kw = k_ref[pl.ds(ws, WKV), h, :]
        vw = v_ref[pl.ds(ws, WKV), h, :]
        kg = k_ref[0:GKV, h, :]
        vg = v_ref[0:GKV, h, :]
        sw = lax.dot_general(
            q, kw, (((1,), (1,)), ((), ())), preferred_element_type=jnp.float32
        ) * SCALE
        ww = jnp.where(mask_w, jnp.exp(sw), 0.0)
        sg = lax.dot_general(
            q, kg, (((1,), (1,)), ((), ())), preferred_element_type=jnp.float32
        ) * SCALE
        wg = jnp.where(mask_g, jnp.exp(sg), 0.0)
        o_ref[:, h, :] = (
            jnp.dot(ww, vw, preferred_element_type=jnp.float32)
            + jnp.dot(wg, vg, preferred_element_type=jnp.float32)
        )
        l_cols.append(
            jnp.sum(ww, axis=1, keepdims=True)
            + jnp.sum(wg, axis=1, keepdims=True)
        )
    l_ref[...] = jnp.concatenate(l_cols, axis=1)


def _glob_body(q_ref, k_ref, v_ref, o_ref, l_ref):
    l_cols = []
    for h in range(HQ):
        q = q_ref[:, h, :]
        k = k_ref[:, h, :]
        v = v_ref[:, h, :]
        s = lax.dot_general(
            q, k, (((1,), (1,)), ((), ())), preferred_element_type=jnp.float32
        ) * SCALE
        w = jnp.exp(s)
        o_ref[:, h, :] = jnp.dot(w, v, preferred_element_type=jnp.float32)
        l_cols.append(jnp.sum(w, axis=1, keepdims=True))
    l_ref[...] = jnp.concatenate(l_cols, axis=1)


def _out_rows(o_rows, l_rows, wo_ref):
    acc = None
    for h in range(HQ):
        ctx = o_rows[:, h, :] / l_rows[:, h:h + 1]
        d = jnp.dot(
            ctx, wo_ref[h * DH:(h + 1) * DH, :],
            preferred_element_type=jnp.float32,
        )
        acc = d if acc is None else acc + d
    return acc


def _comm_body(
    o_ref, og_ref, l_ref, wo_ref, out_ref,
    cl, cg, acc_l, acc_g, sliver, tail,
    ring_send_l, ring_recv_l, ring_send_g, ring_recv_g,
    sliv_send, sliv_recv, bc_send, bc_recv_cw, bc_recv_ccw,
):
    my = lax.axis_index("i")
    left = (my + N_DEV - 1) % N_DEV
    right = (my + 1) % N_DEV

    barrier_sem = pltpu.get_barrier_semaphore()
    for nbr in [left, right]:
        pl.semaphore_signal(
            barrier_sem, inc=1,
            device_id=(nbr,), device_id_type=pl.DeviceIdType.MESH,
        )
    pl.semaphore_wait(barrier_sem, 2)

    sliv_rdma = pltpu.make_async_remote_copy(
        src_ref=o_ref.at[pl.ds(SLIV0, SQ - SLIV0)],
        dst_ref=sliver,
        send_sem=sliv_send, recv_sem=sliv_recv,
        device_id=(0,), device_id_type=pl.DeviceIdType.MESH,
    )

    @pl.when(my == 1)
    def _():
        sliv_rdma.start()

    acc_l[...] = l_ref[...]
    acc_g[...] = og_ref[...]
    cl[0] = l_ref[...]
    cg[0] = og_ref[...]
    for hop in range(N_DEV - 1):
        ss, rs = hop % 2, (hop + 1) % 2
        r_l = pltpu.make_async_remote_copy(
            src_ref=cl.at[ss], dst_ref=cl.at[rs],
            send_sem=ring_send_l.at[ss], recv_sem=ring_recv_l.at[rs],
            device_id=(right,), device_id_type=pl.DeviceIdType.MESH,
        )
        r_g = pltpu.make_async_remote_copy(
            src_ref=cg.at[ss], dst_ref=cg.at[rs],
            send_sem=ring_send_g.at[ss], recv_sem=ring_recv_g.at[rs],
            device_id=(right,), device_id_type=pl.DeviceIdType.MESH,
        )
        r_l.start()
        r_g.start()
        r_l.wait()
        r_g.wait()
        acc_l[...] += cl[rs]
        acc_g[...] += cg[rs]

    out_ref[0:NG, :] = _out_rows(acc_g[...], acc_l[0:NG, :], wo_ref)

    @pl.when(my == 0)
    def _():
        sliv_rdma.wait_recv()
        tail[...] = o_ref[SLIV0:SQ] + sliver[...]
        for c, (r0, nr) in enumerate(CHUNKS):
            if r0 >= SLIV0:
                o_rows = tail[r0 - SLIV0:r0 - SLIV0 + nr]
            else:
                o_rows = o_ref[r0:r0 + nr]
            out_ref[r0:r0 + nr, :] = _out_rows(
                o_rows, acc_l[r0:r0 + nr, :], wo_ref
            )
            tgt = 1 if c % 2 == 0 else 3
            recv_arr = bc_recv_cw if c % 2 == 0 else bc_recv_ccw
            rdma = pltpu.make_async_remote_copy(
                src_ref=out_ref.at[pl.ds(r0, nr)],
                dst_ref=out_ref.at[pl.ds(r0, nr)],
                send_sem=bc_send.at[c], recv_sem=recv_arr.at[c // 2],
                device_id=(tgt,), device_id_type=pl.DeviceIdType.MESH,
            )
            rdma.start()
        for c, (r0, nr) in enumerate(CHUNKS):
            recv_arr = bc_recv_cw if c % 2 == 0 else bc_recv_ccw
            pltpu.make_async_remote_copy(
                src_ref=out_ref.at[pl.ds(r0, nr)],
                dst_ref=out_ref.at[pl.ds(r0, nr)],
                send_sem=bc_send.at[c], recv_sem=recv_arr.at[c // 2],
                device_id=(1 if c % 2 == 0 else 3,),
                device_id_type=pl.DeviceIdType.MESH,
            ).wait_send()

    for par, relay, own_arr, other_arr in (
        (0, 1, bc_recv_cw, bc_recv_ccw),
        (1, 3, bc_recv_ccw, bc_recv_cw),
    ):
        @pl.when(my == relay)
        def _(par=par, own_arr=own_arr, other_arr=other_arr):
            for c, (r0, nr) in enumerate(CHUNKS):
                if c % 2 != par:
                    continue
                rdma = pltpu.make_async_remote_copy(
                    src_ref=out_ref.at[pl.ds(r0, nr)],
                    dst_ref=out_ref.at[pl.ds(r0, nr)],
                    send_sem=bc_send.at[c], recv_sem=own_arr.at[c // 2],
                    device_id=(2,), device_id_type=pl.DeviceIdType.MESH,
                )
                rdma.wait_recv()
                rdma.start()
            for c, (r0, nr) in enumerate(CHUNKS):
                if c % 2 == par:
                    continue
                pltpu.make_async_remote_copy(
                    src_ref=out_ref.at[pl.ds(r0, nr)],
                    dst_ref=out_ref.at[pl.ds(r0, nr)],
                    send_sem=bc_send.at[c], recv_sem=other_arr.at[c // 2],
                    device_id=(2,), device_id_type=pl.DeviceIdType.MESH,
                ).wait_recv()
            for c, (r0, nr) in enumerate(CHUNKS):
                if c % 2 != par:
                    continue
                pltpu.make_async_remote_copy(
                    src_ref=out_ref.at[pl.ds(r0, nr)],
                    dst_ref=out_ref.at[pl.ds(r0, nr)],
                    send_sem=bc_send.at[c], recv_sem=own_arr.at[c // 2],
                    device_id=(2,), device_id_type=pl.DeviceIdType.MESH,
                ).wait_send()

    @pl.when(my == 2)
    def _():
        for c, (r0, nr) in enumerate(CHUNKS):
            recv_arr = bc_recv_cw if c % 2 == 0 else bc_recv_ccw
            tgt = 3 if c % 2 == 0 else 1
            rdma = pltpu.make_async_remote_copy(
                src_ref=out_ref.at[pl.ds(r0, nr)],
                dst_ref=out_ref.at[pl.ds(r0, nr)],
                send_sem=bc_send.at[c], recv_sem=recv_arr.at[c // 2],
                device_id=(tgt,), device_id_type=pl.DeviceIdType.MESH,
            )
            rdma.wait_recv()
            rdma.start()
        for c, (r0, nr) in enumerate(CHUNKS):
            recv_arr = bc_recv_cw if c % 2 == 0 else bc_recv_ccw
            pltpu.make_async_remote_copy(
                src_ref=out_ref.at[pl.ds(r0, nr)],
                dst_ref=out_ref.at[pl.ds(r0, nr)],
                send_sem=bc_send.at[c], recv_sem=recv_arr.at[c // 2],
                device_id=(3 if c % 2 == 0 else 1,),
                device_id_type=pl.DeviceIdType.MESH,
            ).wait_send()

    @pl.when(my == 1)
    def _():
        sliv_rdma.wait_send()

    @functools.partial(
        pl.run_scoped, second_barrier=pltpu.SemaphoreType.REGULAR
    )
    def _(second_barrier):
        for nbr in [left, right]:
            pl.semaphore_signal(
                second_barrier, inc=1,
                device_id=(nbr,), device_id_type=pl.DeviceIdType.MESH,
            )
        pl.semaphore_wait(second_barrier, 2)


def kernel(x, Wq, K_ext, V_ext, Wo):
    x2 = x[0]
    K = K_ext[0]
    V = V_ext[0]

    q = pl.pallas_call(
        _q_body,
        out_shape=jax.ShapeDtypeStruct((SQ, DM), jnp.float32),
        in_specs=[
            pl.BlockSpec(memory_space=pltpu.VMEM),
            pl.BlockSpec(memory_space=pltpu.VMEM),
        ],
        out_specs=pl.BlockSpec(memory_space=pltpu.VMEM),
    )(x2, Wq)
    q3 = q.reshape(SQ, HQ, DH)

    o, l_a = pl.pallas_call(
        _attn_body,
        grid=(SQ // QB,),
        in_specs=[
            pl.BlockSpec((QB, HQ, DH), lambda qb: (qb, 0, 0)),
            pl.BlockSpec((SKV_SHARD, HQ, DH), lambda qb: (0, 0, 0)),
            pl.BlockSpec((SKV_SHARD, HQ, DH), lambda qb: (0, 0, 0)),
        ],
        out_shape=[
            jax.ShapeDtypeStruct((SQ, HQ, DH), jnp.float32),
            jax.ShapeDtypeStruct((SQ, HQ), jnp.float32),
        ],
        out_specs=[
            pl.BlockSpec((QB, HQ, DH), lambda qb: (qb, 0, 0)),
            pl.BlockSpec((QB, HQ), lambda qb: (qb, 0)),
        ],
        compiler_params=pltpu.CompilerParams(
            vmem_limit_bytes=100 * 1024 * 1024,
        ),
    )(q3, K, V)

    og, lg = pl.pallas_call(
        _glob_body,
        out_shape=[
            jax.ShapeDtypeStruct((NG, HQ, DH), jnp.float32),
            jax.ShapeDtypeStruct((NG, HQ), jnp.float32),
        ],
        in_specs=[pl.BlockSpec(memory_space=pltpu.VMEM)] * 3,
        out_specs=[pl.BlockSpec(memory_space=pltpu.VMEM)] * 2,
        compiler_params=pltpu.CompilerParams(
            vmem_limit_bytes=100 * 1024 * 1024,
        ),
    )(q3[0:NG], K, V)

    l_comb = jnp.concatenate([lg, l_a[NG:]], axis=0)

    out = pl.pallas_call(
        _comm_body,
        out_shape=jax.ShapeDtypeStruct((SQ, DM), jnp.float32),
        in_specs=[pl.BlockSpec(memory_space=pltpu.VMEM)] * 4,
        out_specs=pl.BlockSpec(memory_space=pltpu.VMEM),
        scratch_shapes=[
            pltpu.VMEM((2, SQ, HQ), jnp.float32),
            pltpu.VMEM((2, NG, HQ, DH), jnp.float32),
            pltpu.VMEM((SQ, HQ), jnp.float32),
            pltpu.VMEM((NG, HQ, DH), jnp.float32),
            pltpu.VMEM((SQ - SLIV0, HQ, DH), jnp.float32),
            pltpu.VMEM((SQ - SLIV0, HQ, DH), jnp.float32),
            pltpu.SemaphoreType.DMA((2,)),
            pltpu.SemaphoreType.DMA((2,)),
            pltpu.SemaphoreType.DMA((2,)),
            pltpu.SemaphoreType.DMA((2,)),
            pltpu.SemaphoreType.DMA,
            pltpu.SemaphoreType.DMA,
            pltpu.SemaphoreType.DMA((12,)),
            pltpu.SemaphoreType.DMA((6,)),
            pltpu.SemaphoreType.DMA((6,)),
        ],
        compiler_params=pltpu.CompilerParams(
            collective_id=0, vmem_limit_bytes=100 * 1024 * 1024,
        ),
    )(o, og, l_comb, Wo)

    return out.reshape(1, SQ, DM)


# device time: 189078 ns/iter; 2.6517x vs baseline; 1.2378x over previous
import functools

import jax
import jax.numpy as jnp
from jax import lax
from jax.experimental import pallas as pl
from jax.experimental.pallas import tpu as pltpu

N_DEV = 4
SQ = 2048
SKV_SHARD = 2048
HQ = 8
DH = 128
DM = 1024
QB = 256
WKV = 512
GKV = 128
NG = 32
SCALE = 0.08838834764831843
BF = jnp.bfloat16

CHUNKS = [(32 + 176 * c, 176) for c in range(10)] + [(1792, 128), (1920, 128)]
SLIV0 = 1920


def _attn_body(x_ref, wq_ref, k_ref, v_ref, o_ref, l_ref, og_ref, lg_ref):
    my = lax.axis_index("i")
    qb = pl.program_id(0)
    q = lax.dot_general(
        x_ref[...].astype(BF), wq_ref[...].astype(BF),
        (((1,), (0,)), ((), ())), preferred_element_type=jnp.float32,
    ).astype(BF)

    ws = jnp.clip(qb * QB - 128 - my * SKV_SHARD, 0, SKV_SHARD - WKV)
    qi_w = qb * QB + lax.broadcasted_iota(jnp.int32, (QB, WKV), 0)
    ki_w = my * SKV_SHARD + ws + lax.broadcasted_iota(jnp.int32, (QB, WKV), 1)
    mask_w = (jnp.abs(qi_w - ki_w) <= 128) & (ki_w >= NG)
    ki_g = my * SKV_SHARD + lax.broadcasted_iota(jnp.int32, (QB, GKV), 1)
    mask_g = ki_g < NG
    l_cols = []
    for h in range(HQ):
        qh = q[:, h * DH:(h + 1) * DH]
        kw = k_ref[pl.ds(ws, WKV), h, :].astype(BF)
        vw = v_ref[pl.ds(ws, WKV), h, :].astype(BF)
        kg = k_ref[0:GKV, h, :].astype(BF)
        vg = v_ref[0:GKV, h, :].astype(BF)
        sw = lax.dot_general(
            qh, kw, (((1,), (1,)), ((), ())), preferred_element_type=jnp.float32
        ) * SCALE
        ww = jnp.where(mask_w, jnp.exp(sw), 0.0)
        sg = lax.dot_general(
            qh, kg, (((1,), (1,)), ((), ())), preferred_element_type=jnp.float32
        ) * SCALE
        wg = jnp.where(mask_g, jnp.exp(sg), 0.0)
        o_ref[:, h, :] = (
            jnp.dot(ww.astype(BF), vw, preferred_element_type=jnp.float32)
            + jnp.dot(wg.astype(BF), vg, preferred_element_type=jnp.float32)
        )
        l_cols.append(
            jnp.sum(ww, axis=1, keepdims=True)
            + jnp.sum(wg, axis=1, keepdims=True)
        )
    l_ref[...] = jnp.concatenate(l_cols, axis=1)

    @pl.when(qb == 0)
    def _():
        lg_cols = []
        for h in range(HQ):
            q0 = q[0:NG, h * DH:(h + 1) * DH]
            k = k_ref[:, h, :].astype(BF)
            v = v_ref[:, h, :].astype(BF)
            s = lax.dot_general(
                q0, k, (((1,), (1,)), ((), ())),
                preferred_element_type=jnp.float32,
            ) * SCALE
            w = jnp.exp(s)
            og_ref[:, h, :] = jnp.dot(
                w.astype(BF), v, preferred_element_type=jnp.float32
            )
            lg_cols.append(jnp.sum(w, axis=1, keepdims=True))
        lg_ref[...] = jnp.concatenate(lg_cols, axis=1)


def _out_rows(o_rows, l_rows, wo_bf):
    acc = None
    for h in range(HQ):
        ctx = (o_rows[:, h, :] / l_rows[:, h:h + 1]).astype(BF)
        d = jnp.dot(
            ctx, wo_bf[h * DH:(h + 1) * DH, :],
            preferred_element_type=jnp.float32,
        )
        acc = d if acc is None else acc + d
    return acc


def _comm_body(
    o_ref, og_ref, l_ref, lg_ref, wo_ref, out_ref,
    sl_send, aa_l, aa_g, acc_l, sliver, tail, bc_buf,
    aa_send_l, aa_recv_l, aa_send_g, aa_recv_g,
    sliv_send, sliv_recv, bc_send, bc_recv_cw, bc_recv_ccw,
):
    my = lax.axis_index("i")

    barrier_sem = pltpu.get_barrier_semaphore()
    for k in range(1, N_DEV):
        pl.semaphore_signal(
            barrier_sem, inc=1,
            device_id=((my + k) % N_DEV,),
            device_id_type=pl.DeviceIdType.MESH,
        )
    pl.semaphore_wait(barrier_sem, N_DEV - 1)

    sliv_rdma = pltpu.make_async_remote_copy(
        src_ref=o_ref.at[pl.ds(SLIV0, SQ - SLIV0)],
        dst_ref=sliver,
        send_sem=sliv_send, recv_sem=sliv_recv,
        device_id=(0,), device_id_type=pl.DeviceIdType.MESH,
    )

    @pl.when(my == 1)
    def _():
        sliv_rdma.start()

    sl_send[0:NG, :] = lg_ref[...]
    sl_send[NG:SQ, :] = l_ref[NG:SQ, :]
    aa_rdmas = []
    for k in range(1, N_DEV):
        slot = N_DEV - 1 - k
        r_l = pltpu.make_async_remote_copy(
            src_ref=sl_send, dst_ref=aa_l.at[slot],
            send_sem=aa_send_l.at[k - 1], recv_sem=aa_recv_l.at[slot],
            device_id=((my + k) % N_DEV,),
            device_id_type=pl.DeviceIdType.MESH,
        )
        r_g = pltpu.make_async_remote_copy(
            src_ref=og_ref, dst_ref=aa_g.at[slot],
            send_sem=aa_send_g.at[k - 1], recv_sem=aa_recv_g.at[slot],
            device_id=((my + k) % N_DEV,),
            device_id_type=pl.DeviceIdType.MESH,
        )
        r_l.start()
        r_g.start()
        aa_rdmas.append((r_l, r_g))

    wo_bf = wo_ref[...].astype(BF)

    def _wait_aa_and_acc():
        for r_l, r_g in aa_rdmas:
            r_l.wait_recv()
            r_g.wait_recv()
        acc_l[...] = sl_send[...] + aa_l[0] + aa_l[1] + aa_l[2]

    def _out32():
        acc_g = og_ref[...] + aa_g[0] + aa_g[1] + aa_g[2]
        out_ref[0:NG, :] = _out_rows(acc_g, acc_l[0:NG, :], wo_bf)

    @pl.when(my == 0)
    def _():
        for c, (r0, nr) in enumerate(CHUNKS):
            if r0 >= SLIV0:
                sliv_rdma.wait_recv()
                _wait_aa_and_acc()
                tail[...] = o_ref[SLIV0:SQ] + sliver[...]
                o_rows = tail[r0 - SLIV0:r0 - SLIV0 + nr]
                l_rows = acc_l[r0:r0 + nr, :]
            else:
                o_rows = o_ref[r0:r0 + nr]
                l_rows = l_ref[r0:r0 + nr, :]
            rows_f32 = _out_rows(o_rows, l_rows, wo_bf)
            out_ref[r0:r0 + nr, :] = rows_f32
            bc_buf[r0:r0 + nr, :] = rows_f32.astype(BF)
            tgt = 1 if c % 2 == 0 else 3
            recv_arr = bc_recv_cw if c % 2 == 0 else bc_recv_ccw
            pltpu.make_async_remote_copy(
                src_ref=bc_buf.at[pl.ds(r0, nr)],
                dst_ref=bc_buf.at[pl.ds(r0, nr)],
                send_sem=bc_send.at[c], recv_sem=recv_arr.at[c // 2],
                device_id=(tgt,), device_id_type=pl.DeviceIdType.MESH,
            ).start()
        _out32()
        for c, (r0, nr) in enumerate(CHUNKS):
            recv_arr = bc_recv_cw if c % 2 == 0 else bc_recv_ccw
            pltpu.make_async_remote_copy(
                src_ref=bc_buf.at[pl.ds(r0, nr)],
                dst_ref=bc_buf.at[pl.ds(r0, nr)],
                send_sem=bc_send.at[c], recv_sem=recv_arr.at[c // 2],
                device_id=(1 if c % 2 == 0 else 3,),
                device_id_type=pl.DeviceIdType.MESH,
            ).wait_send()

    @pl.when(my != 0)
    def _():
        _wait_aa_and_acc()
        _out32()

    for par, relay, own_arr, other_arr in (
        (0, 1, bc_recv_cw, bc_recv_ccw),
        (1, 3, bc_recv_ccw, bc_recv_cw),
    ):
        @pl.when(my == relay)
        def _(par=par, own_arr=own_arr, other_arr=other_arr):
            for c, (r0, nr) in enumerate(CHUNKS):
                if c % 2 != par:
                    continue
                rdma = pltpu.make_async_remote_copy(
                    src_ref=bc_buf.at[pl.ds(r0, nr)],
                    dst_ref=bc_buf.at[pl.ds(r0, nr)],
                    send_sem=bc_send.at[c], recv_sem=own_arr.at[c // 2],
                    device_id=(2,), device_id_type=pl.DeviceIdType.MESH,
                )
                rdma.wait_recv()
                rdma.start()
                out_ref[r0:r0 + nr, :] = bc_buf[r0:r0 + nr, :].astype(
                    jnp.float32
                )
            for c, (r0, nr) in enumerate(CHUNKS):
                if c % 2 == par:
                    continue
                pltpu.make_async_remote_copy(
                    src_ref=bc_buf.at[pl.ds(r0, nr)],
                    dst_ref=bc_buf.at[pl.ds(r0, nr)],
                    send_sem=bc_send.at[c], recv_sem=other_arr.at[c // 2],
                    device_id=(2,), device_id_type=pl.DeviceIdType.MESH,
                ).wait_recv()
                out_ref[r0:r0 + nr, :] = bc_buf[r0:r0 + nr, :].astype(
                    jnp.float32
                )
            for c, (r0, nr) in enumerate(CHUNKS):
                if c % 2 != par:
                    continue
                pltpu.make_async_remote_copy(
                    src_ref=bc_buf.at[pl.ds(r0, nr)],
                    dst_ref=bc_buf.at[pl.ds(r0, nr)],
                    send_sem=bc_send.at[c], recv_sem=own_arr.at[c // 2],
                    device_id=(2,), device_id_type=pl.DeviceIdType.MESH,
                ).wait_send()

    @pl.when(my == 2)
    def _():
        for c, (r0, nr) in enumerate(CHUNKS):
            recv_arr = bc_recv_cw if c % 2 == 0 else bc_recv_ccw
            tgt = 3 if c % 2 == 0 else 1
            rdma = pltpu.make_async_remote_copy(
                src_ref=bc_buf.at[pl.ds(r0, nr)],
                dst_ref=bc_buf.at[pl.ds(r0, nr)],
                send_sem=bc_send.at[c], recv_sem=recv_arr.at[c // 2],
                device_id=(tgt,), device_id_type=pl.DeviceIdType.MESH,
            )
            rdma.wait_recv()
            rdma.start()
            out_ref[r0:r0 + nr, :] = bc_buf[r0:r0 + nr, :].astype(jnp.float32)
        for c, (r0, nr) in enumerate(CHUNKS):
            recv_arr = bc_recv_cw if c % 2 == 0 else bc_recv_ccw
            pltpu.make_async_remote_copy(
                src_ref=bc_buf.at[pl.ds(r0, nr)],
                dst_ref=bc_buf.at[pl.ds(r0, nr)],
                send_sem=bc_send.at[c], recv_sem=recv_arr.at[c // 2],
                device_id=(3 if c % 2 == 0 else 1,),
                device_id_type=pl.DeviceIdType.MESH,
            ).wait_send()

    for r_l, r_g in aa_rdmas:
        r_l.wait_send()
        r_g.wait_send()

    @pl.when(my == 1)
    def _():
        sliv_rdma.wait_send()

    @functools.partial(
        pl.run_scoped, second_barrier=pltpu.SemaphoreType.REGULAR
    )
    def _(second_barrier):
        for k in range(1, N_DEV):
            pl.semaphore_signal(
                second_barrier, inc=1,
                device_id=((my + k) % N_DEV,),
                device_id_type=pl.DeviceIdType.MESH,
            )
        pl.semaphore_wait(second_barrier, N_DEV - 1)


def kernel(x, Wq, K_ext, V_ext, Wo):
    x2 = x[0]
    K = K_ext[0]
    V = V_ext[0]

    o, l_a, og, lg = pl.pallas_call(
        _attn_body,
        grid=(SQ // QB,),
        in_specs=[
            pl.BlockSpec((QB, DM), lambda qb: (qb, 0)),
            pl.BlockSpec((DM, DM), lambda qb: (0, 0)),
            pl.BlockSpec((SKV_SHARD, HQ, DH), lambda qb: (0, 0, 0)),
            pl.BlockSpec((SKV_SHARD, HQ, DH), lambda qb: (0, 0, 0)),
        ],
        out_shape=[
            jax.ShapeDtypeStruct((SQ, HQ, DH), jnp.float32),
            jax.ShapeDtypeStruct((SQ, HQ), jnp.float32),
            jax.ShapeDtypeStruct((NG, HQ, DH), jnp.float32),
            jax.ShapeDtypeStruct((NG, HQ), jnp.float32),
        ],
        out_specs=[
            pl.BlockSpec((QB, HQ, DH), lambda qb: (qb, 0, 0)),
            pl.BlockSpec((QB, HQ), lambda qb: (qb, 0)),
            pl.BlockSpec((NG, HQ, DH), lambda qb: (0, 0, 0)),
            pl.BlockSpec((NG, HQ), lambda qb: (0, 0)),
        ],
        compiler_params=pltpu.CompilerParams(
            vmem_limit_bytes=100 * 1024 * 1024,
        ),
    )(x2, Wq, K, V)

    out = pl.pallas_call(
        _comm_body,
        out_shape=jax.ShapeDtypeStruct((SQ, DM), jnp.float32),
        in_specs=[pl.BlockSpec(memory_space=pltpu.VMEM)] * 5,
        out_specs=pl.BlockSpec(memory_space=pltpu.VMEM),
        scratch_shapes=[
            pltpu.VMEM((SQ, HQ), jnp.float32),
            pltpu.VMEM((3, SQ, HQ), jnp.float32),
            pltpu.VMEM((3, NG, HQ, DH), jnp.float32),
            pltpu.VMEM((SQ, HQ), jnp.float32),
            pltpu.VMEM((SQ - SLIV0, HQ, DH), jnp.float32),
            pltpu.VMEM((SQ - SLIV0, HQ, DH), jnp.float32),
            pltpu.VMEM((SQ, DM), BF),
            pltpu.SemaphoreType.DMA((3,)),
            pltpu.SemaphoreType.DMA((3,)),
            pltpu.SemaphoreType.DMA((3,)),
            pltpu.SemaphoreType.DMA((3,)),
            pltpu.SemaphoreType.DMA,
            pltpu.SemaphoreType.DMA,
            pltpu.SemaphoreType.DMA((12,)),
            pltpu.SemaphoreType.DMA((6,)),
            pltpu.SemaphoreType.DMA((6,)),
        ],
        compiler_params=pltpu.CompilerParams(
            collective_id=0, vmem_limit_bytes=100 * 1024 * 1024,
        ),
    )(o, og, l_a, lg, Wo)

    return out.reshape(1, SQ, DM)


# device time: 187865 ns/iter; 2.6688x vs baseline; 1.0065x over previous
import functools

import jax
import jax.numpy as jnp
from jax import lax
from jax.experimental import pallas as pl
from jax.experimental.pallas import tpu as pltpu

N_DEV = 4
SQ = 2048
SKV_SHARD = 2048
HQ = 8
DH = 128
DM = 1024
QB = 256
WKV = 512
GKV = 128
NG = 32
SCALE = 0.08838834764831843
BF = jnp.bfloat16

CHUNKS = [(32 + 176 * c, 176) for c in range(10)] + [(1792, 128), (1920, 128)]
SLIV0 = 1920


def _attn_body(x_ref, wq_ref, k_ref, v_ref, o_ref, l_ref, og_ref, lg_ref):
    my = lax.axis_index("i")
    qb = pl.program_id(0)
    q = lax.dot_general(
        x_ref[...], wq_ref[...],
        (((1,), (0,)), ((), ())), preferred_element_type=jnp.float32,
    ).astype(BF)

    ws = jnp.clip(qb * QB - 128 - my * SKV_SHARD, 0, SKV_SHARD - WKV)
    qi_w = qb * QB + lax.broadcasted_iota(jnp.int32, (QB, WKV), 0)
    ki_w = my * SKV_SHARD + ws + lax.broadcasted_iota(jnp.int32, (QB, WKV), 1)
    mask_w = (jnp.abs(qi_w - ki_w) <= 128) & (ki_w >= NG)
    ki_g = my * SKV_SHARD + lax.broadcasted_iota(jnp.int32, (QB, GKV), 1)
    mask_g = ki_g < NG
    l_cols = []
    for h in range(HQ):
        qh = q[:, h * DH:(h + 1) * DH]
        kw = k_ref[pl.ds(ws, WKV), h, :]
        vw = v_ref[pl.ds(ws, WKV), h, :]
        kg = k_ref[0:GKV, h, :]
        vg = v_ref[0:GKV, h, :]
        sw = lax.dot_general(
            qh, kw, (((1,), (1,)), ((), ())), preferred_element_type=jnp.float32
        ) * SCALE
        ww = jnp.where(mask_w, jnp.exp(sw), 0.0)
        sg = lax.dot_general(
            qh, kg, (((1,), (1,)), ((), ())), preferred_element_type=jnp.float32
        ) * SCALE
        wg = jnp.where(mask_g, jnp.exp(sg), 0.0)
        o_ref[:, h, :] = (
            jnp.dot(ww.astype(BF), vw, preferred_element_type=jnp.float32)
            + jnp.dot(wg.astype(BF), vg, preferred_element_type=jnp.float32)
        )
        l_cols.append(
            jnp.sum(ww, axis=1, keepdims=True)
            + jnp.sum(wg, axis=1, keepdims=True)
        )
    l_ref[...] = jnp.concatenate(l_cols, axis=1)

    @pl.when(qb == 0)
    def _():
        lg_cols = []
        for h in range(HQ):
            q0 = q[0:NG, h * DH:(h + 1) * DH]
            k = k_ref[:, h, :]
            v = v_ref[:, h, :]
            s = lax.dot_general(
                q0, k, (((1,), (1,)), ((), ())),
                preferred_element_type=jnp.float32,
            ) * SCALE
            w = jnp.exp(s)
            og_ref[:, h, :] = jnp.dot(
                w.astype(BF), v, preferred_element_type=jnp.float32
            )
            lg_cols.append(jnp.sum(w, axis=1, keepdims=True))
        lg_ref[...] = jnp.concatenate(lg_cols, axis=1)


def _out_rows(o_rows, l_rows, wo_bf):
    acc = None
    for h in range(HQ):
        ctx = (o_rows[:, h, :] / l_rows[:, h:h + 1]).astype(BF)
        d = jnp.dot(
            ctx, wo_bf[h * DH:(h + 1) * DH, :],
            preferred_element_type=jnp.float32,
        )
        acc = d if acc is None else acc + d
    return acc


def _comm_body(
    o_ref, og_ref, l_ref, lg_ref, wo_ref, out_ref,
    sl_send, aa_l, aa_g, acc_l, sliver, tail, bc_buf,
    aa_send_l, aa_recv_l, aa_send_g, aa_recv_g,
    sliv_send, sliv_recv, bc_send, bc_recv_cw, bc_recv_ccw,
):
    my = lax.axis_index("i")

    barrier_sem = pltpu.get_barrier_semaphore()
    for k in range(1, N_DEV):
        pl.semaphore_signal(
            barrier_sem, inc=1,
            device_id=((my + k) % N_DEV,),
            device_id_type=pl.DeviceIdType.MESH,
        )
    pl.semaphore_wait(barrier_sem, N_DEV - 1)

    sliv_rdma = pltpu.make_async_remote_copy(
        src_ref=o_ref.at[pl.ds(SLIV0, SQ - SLIV0)],
        dst_ref=sliver,
        send_sem=sliv_send, recv_sem=sliv_recv,
        device_id=(0,), device_id_type=pl.DeviceIdType.MESH,
    )

    @pl.when(my == 1)
    def _():
        sliv_rdma.start()

    sl_send[0:NG, :] = lg_ref[...]
    sl_send[NG:SQ, :] = l_ref[NG:SQ, :]
    aa_rdmas = []
    for k in range(1, N_DEV):
        slot = N_DEV - 1 - k
        r_l = pltpu.make_async_remote_copy(
            src_ref=sl_send, dst_ref=aa_l.at[slot],
            send_sem=aa_send_l.at[k - 1], recv_sem=aa_recv_l.at[slot],
            device_id=((my + k) % N_DEV,),
            device_id_type=pl.DeviceIdType.MESH,
        )
        r_g = pltpu.make_async_remote_copy(
            src_ref=og_ref, dst_ref=aa_g.at[slot],
            send_sem=aa_send_g.at[k - 1], recv_sem=aa_recv_g.at[slot],
            device_id=((my + k) % N_DEV,),
            device_id_type=pl.DeviceIdType.MESH,
        )
        r_l.start()
        r_g.start()
        aa_rdmas.append((r_l, r_g))

    wo_bf = wo_ref[...]

    def _wait_aa_and_acc():
        for r_l, r_g in aa_rdmas:
            r_l.wait_recv()
            r_g.wait_recv()
        acc_l[...] = sl_send[...] + aa_l[0] + aa_l[1] + aa_l[2]

    def _out32():
        acc_g = og_ref[...] + aa_g[0] + aa_g[1] + aa_g[2]
        out_ref[0:NG, :] = _out_rows(acc_g, acc_l[0:NG, :], wo_bf)

    @pl.when(my == 0)
    def _():
        for c, (r0, nr) in enumerate(CHUNKS):
            if r0 >= SLIV0:
                sliv_rdma.wait_recv()
                _wait_aa_and_acc()
                tail[...] = o_ref[SLIV0:SQ] + sliver[...]
                o_rows = tail[r0 - SLIV0:r0 - SLIV0 + nr]
                l_rows = acc_l[r0:r0 + nr, :]
            else:
                o_rows = o_ref[r0:r0 + nr]
                l_rows = l_ref[r0:r0 + nr, :]
            rows_f32 = _out_rows(o_rows, l_rows, wo_bf)
            out_ref[r0:r0 + nr, :] = rows_f32
            bc_buf[r0:r0 + nr, :] = rows_f32.astype(BF)
            tgt = 1 if c % 2 == 0 else 3
            recv_arr = bc_recv_cw if c % 2 == 0 else bc_recv_ccw
            pltpu.make_async_remote_copy(
                src_ref=bc_buf.at[pl.ds(r0, nr)],
                dst_ref=bc_buf.at[pl.ds(r0, nr)],
                send_sem=bc_send.at[c], recv_sem=recv_arr.at[c // 2],
                device_id=(tgt,), device_id_type=pl.DeviceIdType.MESH,
            ).start()
        _out32()
        for c, (r0, nr) in enumerate(CHUNKS):
            recv_arr = bc_recv_cw if c % 2 == 0 else bc_recv_ccw
            pltpu.make_async_remote_copy(
                src_ref=bc_buf.at[pl.ds(r0, nr)],
                dst_ref=bc_buf.at[pl.ds(r0, nr)],
                send_sem=bc_send.at[c], recv_sem=recv_arr.at[c // 2],
                device_id=(1 if c % 2 == 0 else 3,),
                device_id_type=pl.DeviceIdType.MESH,
            ).wait_send()

    @pl.when(my != 0)
    def _():
        _wait_aa_and_acc()
        _out32()

    for par, relay, own_arr, other_arr in (
        (0, 1, bc_recv_cw, bc_recv_ccw),
        (1, 3, bc_recv_ccw, bc_recv_cw),
    ):
        @pl.when(my == relay)
        def _(par=par, own_arr=own_arr, other_arr=other_arr):
            for c, (r0, nr) in enumerate(CHUNKS):
                if c % 2 != par:
                    continue
                rdma = pltpu.make_async_remote_copy(
                    src_ref=bc_buf.at[pl.ds(r0, nr)],
                    dst_ref=bc_buf.at[pl.ds(r0, nr)],
                    send_sem=bc_send.at[c], recv_sem=own_arr.at[c // 2],
                    device_id=(2,), device_id_type=pl.DeviceIdType.MESH,
                )
                rdma.wait_recv()
                rdma.start()
                out_ref[r0:r0 + nr, :] = bc_buf[r0:r0 + nr, :].astype(
                    jnp.float32
                )
            for c, (r0, nr) in enumerate(CHUNKS):
                if c % 2 == par:
                    continue
                pltpu.make_async_remote_copy(
                    src_ref=bc_buf.at[pl.ds(r0, nr)],
                    dst_ref=bc_buf.at[pl.ds(r0, nr)],
                    send_sem=bc_send.at[c], recv_sem=other_arr.at[c // 2],
                    device_id=(2,), device_id_type=pl.DeviceIdType.MESH,
                ).wait_recv()
                out_ref[r0:r0 + nr, :] = bc_buf[r0:r0 + nr, :].astype(
                    jnp.float32
                )
            for c, (r0, nr) in enumerate(CHUNKS):
                if c % 2 != par:
                    continue
                pltpu.make_async_remote_copy(
                    src_ref=bc_buf.at[pl.ds(r0, nr)],
                    dst_ref=bc_buf.at[pl.ds(r0, nr)],
                    send_sem=bc_send.at[c], recv_sem=own_arr.at[c // 2],
                    device_id=(2,), device_id_type=pl.DeviceIdType.MESH,
                ).wait_send()

    @pl.when(my == 2)
    def _():
        for c, (r0, nr) in enumerate(CHUNKS):
            recv_arr = bc_recv_cw if c % 2 == 0 else bc_recv_ccw
            tgt = 3 if c % 2 == 0 else 1
            rdma = pltpu.make_async_remote_copy(
                src_ref=bc_buf.at[pl.ds(r0, nr)],
                dst_ref=bc_buf.at[pl.ds(r0, nr)],
                send_sem=bc_send.at[c], recv_sem=recv_arr.at[c // 2],
                device_id=(tgt,), device_id_type=pl.DeviceIdType.MESH,
            )
            rdma.wait_recv()
            rdma.start()
            out_ref[r0:r0 + nr, :] = bc_buf[r0:r0 + nr, :].astype(jnp.float32)
        for c, (r0, nr) in enumerate(CHUNKS):
            recv_arr = bc_recv_cw if c % 2 == 0 else bc_recv_ccw
            pltpu.make_async_remote_copy(
                src_ref=bc_buf.at[pl.ds(r0, nr)],
                dst_ref=bc_buf.at[pl.ds(r0, nr)],
                send_sem=bc_send.at[c], recv_sem=recv_arr.at[c // 2],
                device_id=(3 if c % 2 == 0 else 1,),
                device_id_type=pl.DeviceIdType.MESH,
            ).wait_send()

    for r_l, r_g in aa_rdmas:
        r_l.wait_send()
        r_g.wait_send()

    @pl.when(my == 1)
    def _():
        sliv_rdma.wait_send()

    @functools.partial(
        pl.run_scoped, second_barrier=pltpu.SemaphoreType.REGULAR
    )
    def _(second_barrier):
        for k in range(1, N_DEV):
            pl.semaphore_signal(
                second_barrier, inc=1,
                device_id=((my + k) % N_DEV,),
                device_id_type=pl.DeviceIdType.MESH,
            )
        pl.semaphore_wait(second_barrier, N_DEV - 1)


def kernel(x, Wq, K_ext, V_ext, Wo):
    x2 = x[0].astype(BF)
    Wq = Wq.astype(BF)
    K = K_ext[0].astype(BF)
    V = V_ext[0].astype(BF)
    Wo = Wo.astype(BF)

    o, l_a, og, lg = pl.pallas_call(
        _attn_body,
        grid=(SQ // QB,),
        in_specs=[
            pl.BlockSpec((QB, DM), lambda qb: (qb, 0)),
            pl.BlockSpec((DM, DM), lambda qb: (0, 0)),
            pl.BlockSpec((SKV_SHARD, HQ, DH), lambda qb: (0, 0, 0)),
            pl.BlockSpec((SKV_SHARD, HQ, DH), lambda qb: (0, 0, 0)),
        ],
        out_shape=[
            jax.ShapeDtypeStruct((SQ, HQ, DH), jnp.float32),
            jax.ShapeDtypeStruct((SQ, HQ), jnp.float32),
            jax.ShapeDtypeStruct((NG, HQ, DH), jnp.float32),
            jax.ShapeDtypeStruct((NG, HQ), jnp.float32),
        ],
        out_specs=[
            pl.BlockSpec((QB, HQ, DH), lambda qb: (qb, 0, 0)),
            pl.BlockSpec((QB, HQ), lambda qb: (qb, 0)),
            pl.BlockSpec((NG, HQ, DH), lambda qb: (0, 0, 0)),
            pl.BlockSpec((NG, HQ), lambda qb: (0, 0)),
        ],
        compiler_params=pltpu.CompilerParams(
            vmem_limit_bytes=100 * 1024 * 1024,
        ),
    )(x2, Wq, K, V)

    out = pl.pallas_call(
        _comm_body,
        out_shape=jax.ShapeDtypeStruct((SQ, DM), jnp.float32),
        in_specs=[pl.BlockSpec(memory_space=pltpu.VMEM)] * 5,
        out_specs=pl.BlockSpec(memory_space=pltpu.VMEM),
        scratch_shapes=[
            pltpu.VMEM((SQ, HQ), jnp.float32),
            pltpu.VMEM((3, SQ, HQ), jnp.float32),
            pltpu.VMEM((3, NG, HQ, DH), jnp.float32),
            pltpu.VMEM((SQ, HQ), jnp.float32),
            pltpu.VMEM((SQ - SLIV0, HQ, DH), jnp.float32),
            pltpu.VMEM((SQ - SLIV0, HQ, DH), jnp.float32),
            pltpu.VMEM((SQ, DM), BF),
            pltpu.SemaphoreType.DMA((3,)),
            pltpu.SemaphoreType.DMA((3,)),
            pltpu.SemaphoreType.DMA((3,)),
            pltpu.SemaphoreType.DMA((3,)),
            pltpu.SemaphoreType.DMA,
            pltpu.SemaphoreType.DMA,
            pltpu.SemaphoreType.DMA((12,)),
            pltpu.SemaphoreType.DMA((6,)),
            pltpu.SemaphoreType.DMA((6,)),
        ],
        compiler_params=pltpu.CompilerParams(
            collective_id=0, vmem_limit_bytes=100 * 1024 * 1024,
        ),
    )(o, og, l_a, lg, Wo)

    return out.reshape(1, SQ, DM)


# device time: 187336 ns/iter; 2.6763x vs baseline; 1.0028x over previous
import functools

import jax
import jax.numpy as jnp
from jax import lax
from jax.experimental import pallas as pl
from jax.experimental.pallas import tpu as pltpu

N_DEV = 4
SQ = 2048
SKV_SHARD = 2048
HQ = 8
DH = 128
DM = 1024
QB = 256
WKV = 512
GKV = 128
NG = 32
SCALE = 0.08838834764831843
BF = jnp.bfloat16

CHUNKS = [(32 + 176 * c, 176) for c in range(10)] + [(1792, 128), (1920, 128)]
SLIV0 = 1920


def _q_body(x_ref, wq_ref, q_ref):
    q_ref[...] = lax.dot_general(
        x_ref[...], wq_ref[...],
        (((1,), (0,)), ((), ())), preferred_element_type=jnp.float32,
    ).astype(BF)


def _attn_body(q_ref, k_ref, v_ref, o_ref, l_ref):
    my = lax.axis_index("i")
    qb = pl.program_id(0)
    ws = jnp.clip(qb * QB - 128 - my * SKV_SHARD, 0, SKV_SHARD - WKV)
    qi_w = qb * QB + lax.broadcasted_iota(jnp.int32, (QB, WKV), 0)
    ki_w = my * SKV_SHARD + ws + lax.broadcasted_iota(jnp.int32, (QB, WKV), 1)
    mask_w = (jnp.abs(qi_w - ki_w) <= 128) & (ki_w >= NG)
    ki_g = my * SKV_SHARD + lax.broadcasted_iota(jnp.int32, (QB, GKV), 1)
    mask_g = ki_g < NG
    l_cols = []
    for h in range(HQ):
        qh = q_ref[:, h, :]
        kw = k_ref[pl.ds(ws, WKV), h, :]
        vw = v_ref[pl.ds(ws, WKV), h, :]
        kg = k_ref[0:GKV, h, :]
        vg = v_ref[0:GKV, h, :]
        sw = lax.dot_general(
            qh, kw, (((1,), (1,)), ((), ())), preferred_element_type=jnp.float32
        ) * SCALE
        ww = jnp.where(mask_w, jnp.exp(sw), 0.0)
        sg = lax.dot_general(
            qh, kg, (((1,), (1,)), ((), ())), preferred_element_type=jnp.float32
        ) * SCALE
        wg = jnp.where(mask_g, jnp.exp(sg), 0.0)
        o_ref[:, h, :] = (
            jnp.dot(ww.astype(BF), vw, preferred_element_type=jnp.float32)
            + jnp.dot(wg.astype(BF), vg, preferred_element_type=jnp.float32)
        )
        l_cols.append(
            jnp.sum(ww, axis=1, keepdims=True)
            + jnp.sum(wg, axis=1, keepdims=True)
        )
    l_ref[...] = jnp.concatenate(l_cols, axis=1)


def _glob_body(q_ref, k_ref, v_ref, o_ref, l_ref):
    l_cols = []
    for h in range(HQ):
        q0 = q_ref[:, h, :]
        k = k_ref[:, h, :]
        v = v_ref[:, h, :]
        s = lax.dot_general(
            q0, k, (((1,), (1,)), ((), ())),
            preferred_element_type=jnp.float32,
        ) * SCALE
        w = jnp.exp(s)
        o_ref[:, h, :] = jnp.dot(
            w.astype(BF), v, preferred_element_type=jnp.float32
        )
        l_cols.append(jnp.sum(w, axis=1, keepdims=True))
    l_ref[...] = jnp.concatenate(l_cols, axis=1)


def _out_rows(o_rows, l_rows, wo_bf):
    acc = None
    for h in range(HQ):
        ctx = (o_rows[:, h, :] / l_rows[:, h:h + 1]).astype(BF)
        d = jnp.dot(
            ctx, wo_bf[h * DH:(h + 1) * DH, :],
            preferred_element_type=jnp.float32,
        )
        acc = d if acc is None else acc + d
    return acc


def _comm_body(
    o_ref, og_ref, l_ref, lg_ref, wo_ref, out_ref,
    sl_send, aa_l, aa_g, acc_l, sliver, tail, bc_buf,
    aa_send_l, aa_recv_l, aa_send_g, aa_recv_g,
    sliv_send, sliv_recv, bc_send, bc_recv_cw, bc_recv_ccw,
):
    my = lax.axis_index("i")

    barrier_sem = pltpu.get_barrier_semaphore()
    for k in range(1, N_DEV):
        pl.semaphore_signal(
            barrier_sem, inc=1,
            device_id=((my + k) % N_DEV,),
            device_id_type=pl.DeviceIdType.MESH,
        )
    pl.semaphore_wait(barrier_sem, N_DEV - 1)

    sliv_rdma = pltpu.make_async_remote_copy(
        src_ref=o_ref.at[pl.ds(SLIV0, SQ - SLIV0)],
        dst_ref=sliver,
        send_sem=sliv_send, recv_sem=sliv_recv,
        device_id=(0,), device_id_type=pl.DeviceIdType.MESH,
    )

    @pl.when(my == 1)
    def _():
        sliv_rdma.start()

    sl_send[0:NG, :] = lg_ref[...]
    sl_send[NG:SQ, :] = l_ref[NG:SQ, :]
    aa_rdmas = []
    for k in range(1, N_DEV):
        slot = N_DEV - 1 - k
        r_l = pltpu.make_async_remote_copy(
            src_ref=sl_send, dst_ref=aa_l.at[slot],
            send_sem=aa_send_l.at[k - 1], recv_sem=aa_recv_l.at[slot],
            device_id=((my + k) % N_DEV,),
            device_id_type=pl.DeviceIdType.MESH,
        )
        r_g = pltpu.make_async_remote_copy(
            src_ref=og_ref, dst_ref=aa_g.at[slot],
            send_sem=aa_send_g.at[k - 1], recv_sem=aa_recv_g.at[slot],
            device_id=((my + k) % N_DEV,),
            device_id_type=pl.DeviceIdType.MESH,
        )
        r_l.start()
        r_g.start()
        aa_rdmas.append((r_l, r_g))

    wo_bf = wo_ref[...]

    def _wait_aa_and_acc():
        for r_l, r_g in aa_rdmas:
            r_l.wait_recv()
            r_g.wait_recv()
        acc_l[...] = sl_send[...] + aa_l[0] + aa_l[1] + aa_l[2]

    def _out32():
        acc_g = og_ref[...] + aa_g[0] + aa_g[1] + aa_g[2]
        out_ref[0:NG, :] = _out_rows(acc_g, acc_l[0:NG, :], wo_bf)

    @pl.when(my == 0)
    def _():
        for c, (r0, nr) in enumerate(CHUNKS):
            if r0 >= SLIV0:
                sliv_rdma.wait_recv()
                _wait_aa_and_acc()
                tail[...] = o_ref[SLIV0:SQ] + sliver[...]
                o_rows = tail[r0 - SLIV0:r0 - SLIV0 + nr]
                l_rows = acc_l[r0:r0 + nr, :]
            else:
                o_rows = o_ref[r0:r0 + nr]
                l_rows = l_ref[r0:r0 + nr, :]
            rows_f32 = _out_rows(o_rows, l_rows, wo_bf)
            out_ref[r0:r0 + nr, :] = rows_f32
            bc_buf[r0:r0 + nr, :] = rows_f32.astype(BF)
            tgt = 1 if c % 2 == 0 else 3
            recv_arr = bc_recv_cw if c % 2 == 0 else bc_recv_ccw
            pltpu.make_async_remote_copy(
                src_ref=bc_buf.at[pl.ds(r0, nr)],
                dst_ref=bc_buf.at[pl.ds(r0, nr)],
                send_sem=bc_send.at[c], recv_sem=recv_arr.at[c // 2],
                device_id=(tgt,), device_id_type=pl.DeviceIdType.MESH,
            ).start()
        _out32()
        for c, (r0, nr) in enumerate(CHUNKS):
            recv_arr = bc_recv_cw if c % 2 == 0 else bc_recv_ccw
            pltpu.make_async_remote_copy(
                src_ref=bc_buf.at[pl.ds(r0, nr)],
                dst_ref=bc_buf.at[pl.ds(r0, nr)],
                send_sem=bc_send.at[c], recv_sem=recv_arr.at[c // 2],
                device_id=(1 if c % 2 == 0 else 3,),
                device_id_type=pl.DeviceIdType.MESH,
            ).wait_send()

    @pl.when(my != 0)
    def _():
        _wait_aa_and_acc()
        _out32()

    for par, relay, own_arr, other_arr in (
        (0, 1, bc_recv_cw, bc_recv_ccw),
        (1, 3, bc_recv_ccw, bc_recv_cw),
    ):
        @pl.when(my == relay)
        def _(par=par, own_arr=own_arr, other_arr=other_arr):
            for c, (r0, nr) in enumerate(CHUNKS):
                if c % 2 != par:
                    continue
                rdma = pltpu.make_async_remote_copy(
                    src_ref=bc_buf.at[pl.ds(r0, nr)],
                    dst_ref=bc_buf.at[pl.ds(r0, nr)],
                    send_sem=bc_send.at[c], recv_sem=own_arr.at[c // 2],
                    device_id=(2,), device_id_type=pl.DeviceIdType.MESH,
                )
                rdma.wait_recv()
                rdma.start()
                out_ref[r0:r0 + nr, :] = bc_buf[r0:r0 + nr, :].astype(
                    jnp.float32
                )
            for c, (r0, nr) in enumerate(CHUNKS):
                if c % 2 == par:
                    continue
                pltpu.make_async_remote_copy(
                    src_ref=bc_buf.at[pl.ds(r0, nr)],
                    dst_ref=bc_buf.at[pl.ds(r0, nr)],
                    send_sem=bc_send.at[c], recv_sem=other_arr.at[c // 2],
                    device_id=(2,), device_id_type=pl.DeviceIdType.MESH,
                ).wait_recv()
                out_ref[r0:r0 + nr, :] = bc_buf[r0:r0 + nr, :].astype(
                    jnp.float32
                )
            for c, (r0, nr) in enumerate(CHUNKS):
                if c % 2 != par:
                    continue
                pltpu.make_async_remote_copy(
                    src_ref=bc_buf.at[pl.ds(r0, nr)],
                    dst_ref=bc_buf.at[pl.ds(r0, nr)],
                    send_sem=bc_send.at[c], recv_sem=own_arr.at[c // 2],
                    device_id=(2,), device_id_type=pl.DeviceIdType.MESH,
                ).wait_send()

    @pl.when(my == 2)
    def _():
        for c, (r0, nr) in enumerate(CHUNKS):
            recv_arr = bc_recv_cw if c % 2 == 0 else bc_recv_ccw
            tgt = 3 if c % 2 == 0 else 1
            rdma = pltpu.make_async_remote_copy(
                src_ref=bc_buf.at[pl.ds(r0, nr)],
                dst_ref=bc_buf.at[pl.ds(r0, nr)],
                send_sem=bc_send.at[c], recv_sem=recv_arr.at[c // 2],
                device_id=(tgt,), device_id_type=pl.DeviceIdType.MESH,
            )
            rdma.wait_recv()
            rdma.start()
            out_ref[r0:r0 + nr, :] = bc_buf[r0:r0 + nr, :].astype(jnp.float32)
        for c, (r0, nr) in enumerate(CHUNKS):
            recv_arr = bc_recv_cw if c % 2 == 0 else bc_recv_ccw
            pltpu.make_async_remote_copy(
                src_ref=bc_buf.at[pl.ds(r0, nr)],
                dst_ref=bc_buf.at[pl.ds(r0, nr)],
                send_sem=bc_send.at[c], recv_sem=recv_arr.at[c // 2],
                device_id=(3 if c % 2 == 0 else 1,),
                device_id_type=pl.DeviceIdType.MESH,
            ).wait_send()

    for r_l, r_g in aa_rdmas:
        r_l.wait_send()
        r_g.wait_send()

    @pl.when(my == 1)
    def _():
        sliv_rdma.wait_send()

    @functools.partial(
        pl.run_scoped, second_barrier=pltpu.SemaphoreType.REGULAR
    )
    def _(second_barrier):
        for k in range(1, N_DEV):
            pl.semaphore_signal(
                second_barrier, inc=1,
                device_id=((my + k) % N_DEV,),
                device_id_type=pl.DeviceIdType.MESH,
            )
        pl.semaphore_wait(second_barrier, N_DEV - 1)


def kernel(x, Wq, K_ext, V_ext, Wo):
    x2 = x[0].astype(BF)
    Wq = Wq.astype(BF)
    K = K_ext[0].astype(BF)
    V = V_ext[0].astype(BF)
    Wo = Wo.astype(BF)

    q = pl.pallas_call(
        _q_body,
        out_shape=jax.ShapeDtypeStruct((SQ, DM), BF),
        in_specs=[
            pl.BlockSpec(memory_space=pltpu.VMEM),
            pl.BlockSpec(memory_space=pltpu.VMEM),
        ],
        out_specs=pl.BlockSpec(memory_space=pltpu.VMEM),
    )(x2, Wq)
    q3 = q.reshape(SQ, HQ, DH)

    o, l_a = pl.pallas_call(
        _attn_body,
        grid=(SQ // QB,),
        in_specs=[
            pl.BlockSpec((QB, HQ, DH), lambda qb: (qb, 0, 0)),
            pl.BlockSpec((SKV_SHARD, HQ, DH), lambda qb: (0, 0, 0)),
            pl.BlockSpec((SKV_SHARD, HQ, DH), lambda qb: (0, 0, 0)),
        ],
        out_shape=[
            jax.ShapeDtypeStruct((SQ, HQ, DH), jnp.float32),
            jax.ShapeDtypeStruct((SQ, HQ), jnp.float32),
        ],
        out_specs=[
            pl.BlockSpec((QB, HQ, DH), lambda qb: (qb, 0, 0)),
            pl.BlockSpec((QB, HQ), lambda qb: (qb, 0)),
        ],
        compiler_params=pltpu.CompilerParams(
            vmem_limit_bytes=100 * 1024 * 1024,
        ),
    )(q3, K, V)

    og, lg = pl.pallas_call(
        _glob_body,
        out_shape=[
            jax.ShapeDtypeStruct((NG, HQ, DH), jnp.float32),
            jax.ShapeDtypeStruct((NG, HQ), jnp.float32),
        ],
        in_specs=[pl.BlockSpec(memory_space=pltpu.VMEM)] * 3,
        out_specs=[pl.BlockSpec(memory_space=pltpu.VMEM)] * 2,
        compiler_params=pltpu.CompilerParams(
            vmem_limit_bytes=100 * 1024 * 1024,
        ),
    )(q3[0:NG], K, V)

    out = pl.pallas_call(
        _comm_body,
        out_shape=jax.ShapeDtypeStruct((SQ, DM), jnp.float32),
        in_specs=[pl.BlockSpec(memory_space=pltpu.VMEM)] * 5,
        out_specs=pl.BlockSpec(memory_space=pltpu.VMEM),
        scratch_shapes=[
            pltpu.VMEM((SQ, HQ), jnp.float32),
            pltpu.VMEM((3, SQ, HQ), jnp.float32),
            pltpu.VMEM((3, NG, HQ, DH), jnp.float32),
            pltpu.VMEM((SQ, HQ), jnp.float32),
            pltpu.VMEM((SQ - SLIV0, HQ, DH), jnp.float32),
            pltpu.VMEM((SQ - SLIV0, HQ, DH), jnp.float32),
            pltpu.VMEM((SQ, DM), BF),
            pltpu.SemaphoreType.DMA((3,)),
            pltpu.SemaphoreType.DMA((3,)),
            pltpu.SemaphoreType.DMA((3,)),
            pltpu.SemaphoreType.DMA((3,)),
            pltpu.SemaphoreType.DMA,
            pltpu.SemaphoreType.DMA,
            pltpu.SemaphoreType.DMA((12,)),
            pltpu.SemaphoreType.DMA((6,)),
            pltpu.SemaphoreType.DMA((6,)),
        ],
        compiler_params=pltpu.CompilerParams(
            collective_id=0, vmem_limit_bytes=100 * 1024 * 1024,
        ),
    )(o, og, l_a, lg, Wo)

    return out.reshape(1, SQ, DM)


# device time: 134646 ns/iter; 3.7237x vs baseline; 1.3913x over previous
import functools

import jax
import jax.numpy as jnp
from jax import lax
from jax.experimental import pallas as pl
from jax.experimental.pallas import tpu as pltpu

N_DEV = 4
SQ = 2048
SKV_SHARD = 2048
HQ = 8
DH = 128
DM = 1024
QB = 256
WKV = 512
GKV = 128
NG = 32
SCALE = 0.08838834764831843
BF = jnp.bfloat16

CHUNKS = [(32 + 176 * c, 176) for c in range(10)] + [(1792, 128), (1920, 128)]
SLIV0 = 1920


def _q_body(x_ref, wq_ref, q_ref):
    q_ref[...] = lax.dot_general(
        x_ref[...], wq_ref[...],
        (((1,), (0,)), ((), ())), preferred_element_type=jnp.float32,
    ).astype(BF)


def _attn_body(q_ref, k_ref, v_ref, o_ref, l_ref):
    my = lax.axis_index("i")
    qb = pl.program_id(0)
    ws = jnp.clip(qb * QB - 128 - my * SKV_SHARD, 0, SKV_SHARD - WKV)
    ws = pl.multiple_of(ws, 128)
    qi_w = qb * QB + lax.broadcasted_iota(jnp.int32, (QB, WKV), 0)
    ki_w = my * SKV_SHARD + ws + lax.broadcasted_iota(jnp.int32, (QB, WKV), 1)
    mask_w = (jnp.abs(qi_w - ki_w) <= 128) & (ki_w >= NG)
    ki_g = my * SKV_SHARD + lax.broadcasted_iota(jnp.int32, (QB, GKV), 1)
    mask_g = ki_g < NG
    l_cols = []
    for h in range(HQ):
        c0, c1 = h * DH, (h + 1) * DH
        qh = q_ref[:, c0:c1]
        kw = k_ref[pl.ds(ws, WKV), c0:c1]
        vw = v_ref[pl.ds(ws, WKV), c0:c1]
        kg = k_ref[0:GKV, c0:c1]
        vg = v_ref[0:GKV, c0:c1]
        sw = lax.dot_general(
            qh, kw, (((1,), (1,)), ((), ())), preferred_element_type=jnp.float32
        ) * SCALE
        ww = jnp.where(mask_w, jnp.exp(sw), 0.0)
        sg = lax.dot_general(
            qh, kg, (((1,), (1,)), ((), ())), preferred_element_type=jnp.float32
        ) * SCALE
        wg = jnp.where(mask_g, jnp.exp(sg), 0.0)
        o_ref[:, c0:c1] = (
            jnp.dot(ww.astype(BF), vw, preferred_element_type=jnp.float32)
            + jnp.dot(wg.astype(BF), vg, preferred_element_type=jnp.float32)
        )
        l_cols.append(
            jnp.sum(ww, axis=1, keepdims=True)
            + jnp.sum(wg, axis=1, keepdims=True)
        )
    l_ref[...] = jnp.concatenate(l_cols, axis=1)


def _glob_body(q_ref, k_ref, v_ref, o_ref, l_ref):
    l_cols = []
    for h in range(HQ):
        c0, c1 = h * DH, (h + 1) * DH
        q0 = q_ref[:, c0:c1]
        k = k_ref[:, c0:c1]
        v = v_ref[:, c0:c1]
        s = lax.dot_general(
            q0, k, (((1,), (1,)), ((), ())),
            preferred_element_type=jnp.float32,
        ) * SCALE
        w = jnp.exp(s)
        o_ref[:, c0:c1] = jnp.dot(
            w.astype(BF), v, preferred_element_type=jnp.float32
        )
        l_cols.append(jnp.sum(w, axis=1, keepdims=True))
    l_ref[...] = jnp.concatenate(l_cols, axis=1)


def _out_rows(o_rows, l_rows, wo_bf):
    acc = None
    for h in range(HQ):
        ctx = (o_rows[:, h * DH:(h + 1) * DH] / l_rows[:, h:h + 1]).astype(BF)
        d = jnp.dot(
            ctx, wo_bf[h * DH:(h + 1) * DH, :],
            preferred_element_type=jnp.float32,
        )
        acc = d if acc is None else acc + d
    return acc


def _comm_body(
    o_ref, og_ref, l_ref, lg_ref, wo_ref, out_ref,
    sl_send, aa_l, aa_g, acc_l, sliver, tail, bc_buf,
    aa_send_l, aa_recv_l, aa_send_g, aa_recv_g,
    sliv_send, sliv_recv, bc_send, bc_recv_cw, bc_recv_ccw,
):
    my = lax.axis_index("i")

    barrier_sem = pltpu.get_barrier_semaphore()
    for k in range(1, N_DEV):
        pl.semaphore_signal(
            barrier_sem, inc=1,
            device_id=((my + k) % N_DEV,),
            device_id_type=pl.DeviceIdType.MESH,
        )
    pl.semaphore_wait(barrier_sem, N_DEV - 1)

    sliv_rdma = pltpu.make_async_remote_copy(
        src_ref=o_ref.at[pl.ds(SLIV0, SQ - SLIV0)],
        dst_ref=sliver,
        send_sem=sliv_send, recv_sem=sliv_recv,
        device_id=(0,), device_id_type=pl.DeviceIdType.MESH,
    )

    @pl.when(my == 1)
    def _():
        sliv_rdma.start()

    sl_send[0:NG, :] = lg_ref[...]
    sl_send[NG:SQ, :] = l_ref[NG:SQ, :]
    aa_rdmas = []
    for k in range(1, N_DEV):
        slot = N_DEV - 1 - k
        r_l = pltpu.make_async_remote_copy(
            src_ref=sl_send, dst_ref=aa_l.at[slot],
            send_sem=aa_send_l.at[k - 1], recv_sem=aa_recv_l.at[slot],
            device_id=((my + k) % N_DEV,),
            device_id_type=pl.DeviceIdType.MESH,
        )
        r_g = pltpu.make_async_remote_copy(
            src_ref=og_ref, dst_ref=aa_g.at[slot],
            send_sem=aa_send_g.at[k - 1], recv_sem=aa_recv_g.at[slot],
            device_id=((my + k) % N_DEV,),
            device_id_type=pl.DeviceIdType.MESH,
        )
        r_l.start()
        r_g.start()
        aa_rdmas.append((r_l, r_g))

    wo_bf = wo_ref[...]

    def _wait_aa_and_acc():
        for r_l, r_g in aa_rdmas:
            r_l.wait_recv()
            r_g.wait_recv()
        acc_l[...] = sl_send[...] + aa_l[0] + aa_l[1] + aa_l[2]

    def _out32():
        acc_g = og_ref[...] + aa_g[0] + aa_g[1] + aa_g[2]
        out_ref[0:NG, :] = _out_rows(acc_g, acc_l[0:NG, :], wo_bf)

    @pl.when(my == 0)
    def _():
        for c, (r0, nr) in enumerate(CHUNKS):
            if r0 >= SLIV0:
                sliv_rdma.wait_recv()
                _wait_aa_and_acc()
                tail[...] = o_ref[SLIV0:SQ, :] + sliver[...]
                o_rows = tail[r0 - SLIV0:r0 - SLIV0 + nr, :]
                l_rows = acc_l[r0:r0 + nr, :]
            else:
                o_rows = o_ref[r0:r0 + nr, :]
                l_rows = l_ref[r0:r0 + nr, :]
            rows_f32 = _out_rows(o_rows, l_rows, wo_bf)
            out_ref[r0:r0 + nr, :] = rows_f32
            bc_buf[r0:r0 + nr, :] = rows_f32.astype(BF)
            tgt = 1 if c % 2 == 0 else 3
            recv_arr = bc_recv_cw if c % 2 == 0 else bc_recv_ccw
            pltpu.make_async_remote_copy(
                src_ref=bc_buf.at[pl.ds(r0, nr)],
                dst_ref=bc_buf.at[pl.ds(r0, nr)],
                send_sem=bc_send.at[c], recv_sem=recv_arr.at[c // 2],
                device_id=(tgt,), device_id_type=pl.DeviceIdType.MESH,
            ).start()
        _out32()
        for c, (r0, nr) in enumerate(CHUNKS):
            recv_arr = bc_recv_cw if c % 2 == 0 else bc_recv_ccw
            pltpu.make_async_remote_copy(
                src_ref=bc_buf.at[pl.ds(r0, nr)],
                dst_ref=bc_buf.at[pl.ds(r0, nr)],
                send_sem=bc_send.at[c], recv_sem=recv_arr.at[c // 2],
                device_id=(1 if c % 2 == 0 else 3,),
                device_id_type=pl.DeviceIdType.MESH,
            ).wait_send()

    @pl.when(my != 0)
    def _():
        _wait_aa_and_acc()
        _out32()

    for par, relay, own_arr, other_arr in (
        (0, 1, bc_recv_cw, bc_recv_ccw),
        (1, 3, bc_recv_ccw, bc_recv_cw),
    ):
        @pl.when(my == relay)
        def _(par=par, own_arr=own_arr, other_arr=other_arr):
            for c, (r0, nr) in enumerate(CHUNKS):
                if c % 2 != par:
                    continue
                rdma = pltpu.make_async_remote_copy(
                    src_ref=bc_buf.at[pl.ds(r0, nr)],
                    dst_ref=bc_buf.at[pl.ds(r0, nr)],
                    send_sem=bc_send.at[c], recv_sem=own_arr.at[c // 2],
                    device_id=(2,), device_id_type=pl.DeviceIdType.MESH,
                )
                rdma.wait_recv()
                rdma.start()
                out_ref[r0:r0 + nr, :] = bc_buf[r0:r0 + nr, :].astype(
                    jnp.float32
                )
            for c, (r0, nr) in enumerate(CHUNKS):
                if c % 2 == par:
                    continue
                pltpu.make_async_remote_copy(
                    src_ref=bc_buf.at[pl.ds(r0, nr)],
                    dst_ref=bc_buf.at[pl.ds(r0, nr)],
                    send_sem=bc_send.at[c], recv_sem=other_arr.at[c // 2],
                    device_id=(2,), device_id_type=pl.DeviceIdType.MESH,
                ).wait_recv()
                out_ref[r0:r0 + nr, :] = bc_buf[r0:r0 + nr, :].astype(
                    jnp.float32
                )
            for c, (r0, nr) in enumerate(CHUNKS):
                if c % 2 != par:
                    continue
                pltpu.make_async_remote_copy(
                    src_ref=bc_buf.at[pl.ds(r0, nr)],
                    dst_ref=bc_buf.at[pl.ds(r0, nr)],
                    send_sem=bc_send.at[c], recv_sem=own_arr.at[c // 2],
                    device_id=(2,), device_id_type=pl.DeviceIdType.MESH,
                ).wait_send()

    @pl.when(my == 2)
    def _():
        for c, (r0, nr) in enumerate(CHUNKS):
            recv_arr = bc_recv_cw if c % 2 == 0 else bc_recv_ccw
            tgt = 3 if c % 2 == 0 else 1
            rdma = pltpu.make_async_remote_copy(
                src_ref=bc_buf.at[pl.ds(r0, nr)],
                dst_ref=bc_buf.at[pl.ds(r0, nr)],
                send_sem=bc_send.at[c], recv_sem=recv_arr.at[c // 2],
                device_id=(tgt,), device_id_type=pl.DeviceIdType.MESH,
            )
            rdma.wait_recv()
            rdma.start()
            out_ref[r0:r0 + nr, :] = bc_buf[r0:r0 + nr, :].astype(jnp.float32)
        for c, (r0, nr) in enumerate(CHUNKS):
            recv_arr = bc_recv_cw if c % 2 == 0 else bc_recv_ccw
            pltpu.make_async_remote_copy(
                src_ref=bc_buf.at[pl.ds(r0, nr)],
                dst_ref=bc_buf.at[pl.ds(r0, nr)],
                send_sem=bc_send.at[c], recv_sem=recv_arr.at[c // 2],
                device_id=(3 if c % 2 == 0 else 1,),
                device_id_type=pl.DeviceIdType.MESH,
            ).wait_send()

    for r_l, r_g in aa_rdmas:
        r_l.wait_send()
        r_g.wait_send()

    @pl.when(my == 1)
    def _():
        sliv_rdma.wait_send()

    @functools.partial(
        pl.run_scoped, second_barrier=pltpu.SemaphoreType.REGULAR
    )
    def _(second_barrier):
        for k in range(1, N_DEV):
            pl.semaphore_signal(
                second_barrier, inc=1,
                device_id=((my + k) % N_DEV,),
                device_id_type=pl.DeviceIdType.MESH,
            )
        pl.semaphore_wait(second_barrier, N_DEV - 1)


def kernel(x, Wq, K_ext, V_ext, Wo):
    x2 = x[0].astype(BF)
    Wq = Wq.astype(BF)
    K = K_ext[0].reshape(SKV_SHARD, DM).astype(BF)
    V = V_ext[0].reshape(SKV_SHARD, DM).astype(BF)
    Wo = Wo.astype(BF)

    q = pl.pallas_call(
        _q_body,
        out_shape=jax.ShapeDtypeStruct((SQ, DM), BF),
        in_specs=[
            pl.BlockSpec(memory_space=pltpu.VMEM),
            pl.BlockSpec(memory_space=pltpu.VMEM),
        ],
        out_specs=pl.BlockSpec(memory_space=pltpu.VMEM),
    )(x2, Wq)

    o, l_a = pl.pallas_call(
        _attn_body,
        grid=(SQ // QB,),
        in_specs=[
            pl.BlockSpec((QB, DM), lambda qb: (qb, 0)),
            pl.BlockSpec((SKV_SHARD, DM), lambda qb: (0, 0)),
            pl.BlockSpec((SKV_SHARD, DM), lambda qb: (0, 0)),
        ],
        out_shape=[
            jax.ShapeDtypeStruct((SQ, DM), jnp.float32),
            jax.ShapeDtypeStruct((SQ, HQ), jnp.float32),
        ],
        out_specs=[
            pl.BlockSpec((QB, DM), lambda qb: (qb, 0)),
            pl.BlockSpec((QB, HQ), lambda qb: (qb, 0)),
        ],
        compiler_params=pltpu.CompilerParams(
            vmem_limit_bytes=100 * 1024 * 1024,
        ),
    )(q, K, V)

    og, lg = pl.pallas_call(
        _glob_body,
        out_shape=[
            jax.ShapeDtypeStruct((NG, DM), jnp.float32),
            jax.ShapeDtypeStruct((NG, HQ), jnp.float32),
        ],
        in_specs=[pl.BlockSpec(memory_space=pltpu.VMEM)] * 3,
        out_specs=[pl.BlockSpec(memory_space=pltpu.VMEM)] * 2,
        compiler_params=pltpu.CompilerParams(
            vmem_limit_bytes=100 * 1024 * 1024,
        ),
    )(q[0:NG], K, V)

    out = pl.pallas_call(
        _comm_body,
        out_shape=jax.ShapeDtypeStruct((SQ, DM), jnp.float32),
        in_specs=[pl.BlockSpec(memory_space=pltpu.VMEM)] * 5,
        out_specs=pl.BlockSpec(memory_space=pltpu.VMEM),
        scratch_shapes=[
            pltpu.VMEM((SQ, HQ), jnp.float32),
            pltpu.VMEM((3, SQ, HQ), jnp.float32),
            pltpu.VMEM((3, NG, DM), jnp.float32),
            pltpu.VMEM((SQ, HQ), jnp.float32),
            pltpu.VMEM((SQ - SLIV0, DM), jnp.float32),
            pltpu.VMEM((SQ - SLIV0, DM), jnp.float32),
            pltpu.VMEM((SQ, DM), BF),
            pltpu.SemaphoreType.DMA((3,)),
            pltpu.SemaphoreType.DMA((3,)),
            pltpu.SemaphoreType.DMA((3,)),
            pltpu.SemaphoreType.DMA((3,)),
            pltpu.SemaphoreType.DMA,
            pltpu.SemaphoreType.DMA,
            pltpu.SemaphoreType.DMA((12,)),
            pltpu.SemaphoreType.DMA((6,)),
            pltpu.SemaphoreType.DMA((6,)),
        ],
        compiler_params=pltpu.CompilerParams(
            collective_id=0, vmem_limit_bytes=100 * 1024 * 1024,
        ),
    )(o, og, l_a, lg, Wo)

    return out.reshape(1, SQ, DM)


# device time: 132415 ns/iter; 3.7864x vs baseline; 1.0168x over previous
import functools

import jax
import jax.numpy as jnp
from jax import lax
from jax.experimental import pallas as pl
from jax.experimental.pallas import tpu as pltpu

N_DEV = 4
SQ = 2048
SKV_SHARD = 2048
HQ = 8
DH = 128
DM = 1024
QB = 256
WKV = 512
GKV = 128
NG = 32
SCALE = 0.08838834764831843
BF = jnp.bfloat16

CHUNKS = [(32 + 176 * c, 176) for c in range(10)] + [(1792, 128), (1920, 128)]
SLIV0 = 1920


def _q_body(x_ref, wq_ref, q_ref):
    q_ref[...] = lax.dot_general(
        x_ref[...], wq_ref[...],
        (((1,), (0,)), ((), ())), preferred_element_type=jnp.float32,
    ).astype(BF)


def _attn_body(q_ref, k_ref, v_ref, o_ref, l_ref):
    my = lax.axis_index("i")
    qb = pl.program_id(0)
    ws = jnp.clip(qb * QB - 128 - my * SKV_SHARD, 0, SKV_SHARD - WKV)
    ws = pl.multiple_of(ws, 128)
    qi_w = qb * QB + lax.broadcasted_iota(jnp.int32, (QB, WKV), 0)
    ki_w = my * SKV_SHARD + ws + lax.broadcasted_iota(jnp.int32, (QB, WKV), 1)
    mask_w = (jnp.abs(qi_w - ki_w) <= 128) & (ki_w >= NG)
    ki_g = my * SKV_SHARD + lax.broadcasted_iota(jnp.int32, (QB, GKV), 1)
    mask_g = ki_g < NG
    l_cols = []
    for h in range(HQ):
        c0, c1 = h * DH, (h + 1) * DH
        qh = q_ref[:, c0:c1]
        kw = k_ref[pl.ds(ws, WKV), c0:c1]
        vw = v_ref[pl.ds(ws, WKV), c0:c1]
        kg = k_ref[0:GKV, c0:c1]
        vg = v_ref[0:GKV, c0:c1]
        sw = lax.dot_general(
            qh, kw, (((1,), (1,)), ((), ())), preferred_element_type=jnp.float32
        ) * SCALE
        ww = jnp.where(mask_w, jnp.exp(sw), 0.0)
        sg = lax.dot_general(
            qh, kg, (((1,), (1,)), ((), ())), preferred_element_type=jnp.float32
        ) * SCALE
        wg = jnp.where(mask_g, jnp.exp(sg), 0.0)
        o_ref[:, c0:c1] = (
            jnp.dot(ww.astype(BF), vw, preferred_element_type=jnp.float32)
            + jnp.dot(wg.astype(BF), vg, preferred_element_type=jnp.float32)
        ).astype(BF)
        l_cols.append(
            jnp.sum(ww, axis=1, keepdims=True)
            + jnp.sum(wg, axis=1, keepdims=True)
        )
    l_ref[...] = jnp.concatenate(l_cols, axis=1)


def _glob_body(q_ref, k_ref, v_ref, o_ref, l_ref):
    l_cols = []
    for h in range(HQ):
        c0, c1 = h * DH, (h + 1) * DH
        q0 = q_ref[:, c0:c1]
        k = k_ref[:, c0:c1]
        v = v_ref[:, c0:c1]
        s = lax.dot_general(
            q0, k, (((1,), (1,)), ((), ())),
            preferred_element_type=jnp.float32,
        ) * SCALE
        w = jnp.exp(s)
        o_ref[:, c0:c1] = jnp.dot(
            w.astype(BF), v, preferred_element_type=jnp.float32
        ).astype(BF)
        l_cols.append(jnp.sum(w, axis=1, keepdims=True))
    l_ref[...] = jnp.concatenate(l_cols, axis=1)


def _out_rows(o_rows, l_rows, wo_bf):
    acc = None
    for h in range(HQ):
        ctx = (
            o_rows[:, h * DH:(h + 1) * DH].astype(jnp.float32)
            / l_rows[:, h:h + 1]
        ).astype(BF)
        d = jnp.dot(
            ctx, wo_bf[h * DH:(h + 1) * DH, :],
            preferred_element_type=jnp.float32,
        )
        acc = d if acc is None else acc + d
    return acc


def _comm_body(
    o_ref, og_ref, l_ref, lg_ref, wo_ref, out_ref,
    sl_send, aa_l, aa_g, acc_l, sliver, tail, bc_buf,
    aa_send_l, aa_recv_l, aa_send_g, aa_recv_g,
    sliv_send, sliv_recv, bc_send, bc_recv_cw, bc_recv_ccw,
):
    my = lax.axis_index("i")

    barrier_sem = pltpu.get_barrier_semaphore()
    for k in range(1, N_DEV):
        pl.semaphore_signal(
            barrier_sem, inc=1,
            device_id=((my + k) % N_DEV,),
            device_id_type=pl.DeviceIdType.MESH,
        )
    pl.semaphore_wait(barrier_sem, N_DEV - 1)

    sliv_rdma = pltpu.make_async_remote_copy(
        src_ref=o_ref.at[pl.ds(SLIV0, SQ - SLIV0)],
        dst_ref=sliver,
        send_sem=sliv_send, recv_sem=sliv_recv,
        device_id=(0,), device_id_type=pl.DeviceIdType.MESH,
    )

    @pl.when(my == 1)
    def _():
        sliv_rdma.start()

    sl_send[0:NG, :] = lg_ref[...]
    sl_send[NG:SQ, :] = l_ref[NG:SQ, :]
    aa_rdmas = []
    for k in range(1, N_DEV):
        slot = N_DEV - 1 - k
        r_l = pltpu.make_async_remote_copy(
            src_ref=sl_send, dst_ref=aa_l.at[slot],
            send_sem=aa_send_l.at[k - 1], recv_sem=aa_recv_l.at[slot],
            device_id=((my + k) % N_DEV,),
            device_id_type=pl.DeviceIdType.MESH,
        )
        r_g = pltpu.make_async_remote_copy(
            src_ref=og_ref, dst_ref=aa_g.at[slot],
            send_sem=aa_send_g.at[k - 1], recv_sem=aa_recv_g.at[slot],
            device_id=((my + k) % N_DEV,),
            device_id_type=pl.DeviceIdType.MESH,
        )
        r_l.start()
        r_g.start()
        aa_rdmas.append((r_l, r_g))

    wo_bf = wo_ref[...]

    def _wait_aa_and_acc():
        for r_l, r_g in aa_rdmas:
            r_l.wait_recv()
            r_g.wait_recv()
        acc_l[...] = sl_send[...] + aa_l[0] + aa_l[1] + aa_l[2]

    def _out32():
        acc_g = (
            og_ref[...].astype(jnp.float32)
            + aa_g[0].astype(jnp.float32)
            + aa_g[1].astype(jnp.float32)
            + aa_g[2].astype(jnp.float32)
        )
        out_ref[0:NG, :] = _out_rows(acc_g, acc_l[0:NG, :], wo_bf)

    @pl.when(my == 0)
    def _():
        for c, (r0, nr) in enumerate(CHUNKS):
            if r0 >= SLIV0:
                sliv_rdma.wait_recv()
                _wait_aa_and_acc()
                tail[...] = (
                    o_ref[SLIV0:SQ, :].astype(jnp.float32)
                    + sliver[...].astype(jnp.float32)
                )
                o_rows = tail[r0 - SLIV0:r0 - SLIV0 + nr, :]
                l_rows = acc_l[r0:r0 + nr, :]
            else:
                o_rows = o_ref[r0:r0 + nr, :]
                l_rows = l_ref[r0:r0 + nr, :]
            rows_f32 = _out_rows(o_rows, l_rows, wo_bf)
            out_ref[r0:r0 + nr, :] = rows_f32
            bc_buf[r0:r0 + nr, :] = rows_f32.astype(BF)
            tgt = 1 if c % 2 == 0 else 3
            recv_arr = bc_recv_cw if c % 2 == 0 else bc_recv_ccw
            pltpu.make_async_remote_copy(
                src_ref=bc_buf.at[pl.ds(r0, nr)],
                dst_ref=bc_buf.at[pl.ds(r0, nr)],
                send_sem=bc_send.at[c], recv_sem=recv_arr.at[c // 2],
                device_id=(tgt,), device_id_type=pl.DeviceIdType.MESH,
            ).start()
        _out32()
        for c, (r0, nr) in enumerate(CHUNKS):
            recv_arr = bc_recv_cw if c % 2 == 0 else bc_recv_ccw
            pltpu.make_async_remote_copy(
                src_ref=bc_buf.at[pl.ds(r0, nr)],
                dst_ref=bc_buf.at[pl.ds(r0, nr)],
                send_sem=bc_send.at[c], recv_sem=recv_arr.at[c // 2],
                device_id=(1 if c % 2 == 0 else 3,),
                device_id_type=pl.DeviceIdType.MESH,
            ).wait_send()

    @pl.when(my != 0)
    def _():
        _wait_aa_and_acc()
        _out32()

    for par, relay, own_arr, other_arr in (
        (0, 1, bc_recv_cw, bc_recv_ccw),
        (1, 3, bc_recv_ccw, bc_recv_cw),
    ):
        @pl.when(my == relay)
        def _(par=par, own_arr=own_arr, other_arr=other_arr):
            for c, (r0, nr) in enumerate(CHUNKS):
                if c % 2 != par:
                    continue
                rdma = pltpu.make_async_remote_copy(
                    src_ref=bc_buf.at[pl.ds(r0, nr)],
                    dst_ref=bc_buf.at[pl.ds(r0, nr)],
                    send_sem=bc_send.at[c], recv_sem=own_arr.at[c // 2],
                    device_id=(2,), device_id_type=pl.DeviceIdType.MESH,
                )
                rdma.wait_recv()
                rdma.start()
                out_ref[r0:r0 + nr, :] = bc_buf[r0:r0 + nr, :].astype(
                    jnp.float32
                )
            for c, (r0, nr) in enumerate(CHUNKS):
                if c % 2 == par:
                    continue
                pltpu.make_async_remote_copy(
                    src_ref=bc_buf.at[pl.ds(r0, nr)],
                    dst_ref=bc_buf.at[pl.ds(r0, nr)],
                    send_sem=bc_send.at[c], recv_sem=other_arr.at[c // 2],
                    device_id=(2,), device_id_type=pl.DeviceIdType.MESH,
                ).wait_recv()
                out_ref[r0:r0 + nr, :] = bc_buf[r0:r0 + nr, :].astype(
                    jnp.float32
                )
            for c, (r0, nr) in enumerate(CHUNKS):
                if c % 2 != par:
                    continue
                pltpu.make_async_remote_copy(
                    src_ref=bc_buf.at[pl.ds(r0, nr)],
                    dst_ref=bc_buf.at[pl.ds(r0, nr)],
                    send_sem=bc_send.at[c], recv_sem=own_arr.at[c // 2],
                    device_id=(2,), device_id_type=pl.DeviceIdType.MESH,
                ).wait_send()

    @pl.when(my == 2)
    def _():
        for c, (r0, nr) in enumerate(CHUNKS):
            recv_arr = bc_recv_cw if c % 2 == 0 else bc_recv_ccw
            tgt = 3 if c % 2 == 0 else 1
            rdma = pltpu.make_async_remote_copy(
                src_ref=bc_buf.at[pl.ds(r0, nr)],
                dst_ref=bc_buf.at[pl.ds(r0, nr)],
                send_sem=bc_send.at[c], recv_sem=recv_arr.at[c // 2],
                device_id=(tgt,), device_id_type=pl.DeviceIdType.MESH,
            )
            rdma.wait_recv()
            rdma.start()
            out_ref[r0:r0 + nr, :] = bc_buf[r0:r0 + nr, :].astype(jnp.float32)
        for c, (r0, nr) in enumerate(CHUNKS):
            recv_arr = bc_recv_cw if c % 2 == 0 else bc_recv_ccw
            pltpu.make_async_remote_copy(
                src_ref=bc_buf.at[pl.ds(r0, nr)],
                dst_ref=bc_buf.at[pl.ds(r0, nr)],
                send_sem=bc_send.at[c], recv_sem=recv_arr.at[c // 2],
                device_id=(3 if c % 2 == 0 else 1,),
                device_id_type=pl.DeviceIdType.MESH,
            ).wait_send()

    for r_l, r_g in aa_rdmas:
        r_l.wait_send()
        r_g.wait_send()

    @pl.when(my == 1)
    def _():
        sliv_rdma.wait_send()

    @functools.partial(
        pl.run_scoped, second_barrier=pltpu.SemaphoreType.REGULAR
    )
    def _(second_barrier):
        for k in range(1, N_DEV):
            pl.semaphore_signal(
                second_barrier, inc=1,
                device_id=((my + k) % N_DEV,),
                device_id_type=pl.DeviceIdType.MESH,
            )
        pl.semaphore_wait(second_barrier, N_DEV - 1)


def kernel(x, Wq, K_ext, V_ext, Wo):
    x2 = x[0].astype(BF)
    Wq = Wq.astype(BF)
    K = K_ext[0].reshape(SKV_SHARD, DM).astype(BF)
    V = V_ext[0].reshape(SKV_SHARD, DM).astype(BF)
    Wo = Wo.astype(BF)

    q = pl.pallas_call(
        _q_body,
        out_shape=jax.ShapeDtypeStruct((SQ, DM), BF),
        in_specs=[
            pl.BlockSpec(memory_space=pltpu.VMEM),
            pl.BlockSpec(memory_space=pltpu.VMEM),
        ],
        out_specs=pl.BlockSpec(memory_space=pltpu.VMEM),
    )(x2, Wq)

    o, l_a = pl.pallas_call(
        _attn_body,
        grid=(SQ // QB,),
        in_specs=[
            pl.BlockSpec((QB, DM), lambda qb: (qb, 0)),
            pl.BlockSpec((SKV_SHARD, DM), lambda qb: (0, 0)),
            pl.BlockSpec((SKV_SHARD, DM), lambda qb: (0, 0)),
        ],
        out_shape=[
            jax.ShapeDtypeStruct((SQ, DM), BF),
            jax.ShapeDtypeStruct((SQ, HQ), jnp.float32),
        ],
        out_specs=[
            pl.BlockSpec((QB, DM), lambda qb: (qb, 0)),
            pl.BlockSpec((QB, HQ), lambda qb: (qb, 0)),
        ],
        compiler_params=pltpu.CompilerParams(
            vmem_limit_bytes=100 * 1024 * 1024,
        ),
    )(q, K, V)

    og, lg = pl.pallas_call(
        _glob_body,
        out_shape=[
            jax.ShapeDtypeStruct((NG, DM), BF),
            jax.ShapeDtypeStruct((NG, HQ), jnp.float32),
        ],
        in_specs=[pl.BlockSpec(memory_space=pltpu.VMEM)] * 3,
        out_specs=[pl.BlockSpec(memory_space=pltpu.VMEM)] * 2,
        compiler_params=pltpu.CompilerParams(
            vmem_limit_bytes=100 * 1024 * 1024,
        ),
    )(q[0:NG], K, V)

    out = pl.pallas_call(
        _comm_body,
        out_shape=jax.ShapeDtypeStruct((SQ, DM), jnp.float32),
        in_specs=[pl.BlockSpec(memory_space=pltpu.VMEM)] * 5,
        out_specs=pl.BlockSpec(memory_space=pltpu.VMEM),
        scratch_shapes=[
            pltpu.VMEM((SQ, HQ), jnp.float32),
            pltpu.VMEM((3, SQ, HQ), jnp.float32),
            pltpu.VMEM((3, NG, DM), BF),
            pltpu.VMEM((SQ, HQ), jnp.float32),
            pltpu.VMEM((SQ - SLIV0, DM), BF),
            pltpu.VMEM((SQ - SLIV0, DM), jnp.float32),
            pltpu.VMEM((SQ, DM), BF),
            pltpu.SemaphoreType.DMA((3,)),
            pltpu.SemaphoreType.DMA((3,)),
            pltpu.SemaphoreType.DMA((3,)),
            pltpu.SemaphoreType.DMA((3,)),
            pltpu.SemaphoreType.DMA,
            pltpu.SemaphoreType.DMA,
            pltpu.SemaphoreType.DMA((12,)),
            pltpu.SemaphoreType.DMA((6,)),
            pltpu.SemaphoreType.DMA((6,)),
        ],
        compiler_params=pltpu.CompilerParams(
            collective_id=0, vmem_limit_bytes=100 * 1024 * 1024,
        ),
    )(o, og, l_a, lg, Wo)

    return out.reshape(1, SQ, DM)
